# Initial kernel scaffold; baseline (speedup 1.0000x reference)
import functools

import jax
import jax.numpy as jnp
from jax import lax
from jax.experimental import pallas as pl
from jax.experimental.pallas import tpu as pltpu
from jax.experimental.pallas import tpu_sc as plsc

_N = 50000
_E = 800000
_D = 64
_W = 80
_EBLK = 4000
_NBLK = 2000
_NC = 2
_NS = 16
_HALF = 25000
_ACCR = 25024
_ZROWS = 391
_GOPS = (2 * _E) // 128
_GPW = (_GOPS + 31) // 32

_f32 = jnp.float32


def _sc_gather(table, idx):
    mesh = plsc.VectorSubcoreMesh(core_axis_name="c", subcore_axis_name="s")

    @functools.partial(
        pl.kernel,
        out_type=jax.ShapeDtypeStruct((2 * _E, _W), _f32),
        mesh=mesh,
        scratch_types=[
            pltpu.VMEM((128,), jnp.int32),
            pltpu.VMEM((128, _W), _f32),
            pltpu.SemaphoreType.DMA,
        ],
    )
    def gk(table_hbm, idx_hbm, out_hbm, idx_v, rows_v, sem):
        wid = lax.axis_index("s") * _NC + lax.axis_index("c")

        def body(i, carry):
            oprow = wid * _GPW + i

            @pl.when(oprow < _GOPS)
            def _():
                base = oprow * 128
                pltpu.sync_copy(idx_hbm.at[pl.ds(base, 128)], idx_v)
                pltpu.async_copy(table_hbm.at[idx_v], rows_v, sem).wait()
                pltpu.sync_copy(rows_v, out_hbm.at[pl.ds(base, 128)])

            return carry

        lax.fori_loop(0, _GPW, body, 0)

    return gk(table, idx)


def _sc_scatter(payload, dst):
    mesh = plsc.VectorSubcoreMesh(core_axis_name="c", subcore_axis_name="s")

    @functools.partial(
        pl.kernel,
        out_type=jax.ShapeDtypeStruct((2 * _ACCR, _W), _f32),
        mesh=mesh,
        scratch_types=[
            pltpu.VMEM((80,), jnp.int32),
            pltpu.VMEM((8, 80), jnp.int32),
            pltpu.VMEM((80, _W), _f32),
            pltpu.VMEM((_ZROWS, _W), _f32),
            pltpu.VMEM_SHARED((_ACCR, _W), _f32),
        ],
    )
    def sk(pay_hbm, dst_hbm, out_hbm, dst_v, lidx, rows_v, zbuf, acc):
        c = lax.axis_index("c")
        s = lax.axis_index("s")
        lo = c * _HALF
        hi = lo + _HALF

        def zrow(rr, carry):
            for k in range(_W // 16):
                zbuf[rr, pl.ds(k * 16, 16)] = jnp.zeros((16,), _f32)
            return carry

        lax.fori_loop(0, _ZROWS, zrow, 0)
        for k in range(4):
            pltpu.sync_copy(
                zbuf, acc.at[pl.ds(s * (4 * _ZROWS) + k * _ZROWS, _ZROWS)])
        plsc.subcore_barrier()

        tbase = s * (_E // _NS)
        iota = lax.broadcasted_iota(jnp.int32, (16,), 0)

        def chunk(i, carry):
            base = tbase + i * 80
            pltpu.sync_copy(dst_hbm.at[pl.ds(base, 80)], dst_v)
            for k in range(5):
                v = dst_v[pl.ds(k * 16, 16)]
                m = (v >= lo) & (v < hi)
                li = jnp.where(m, v - lo, _HALF + (iota & 7))
                lidx[0, pl.ds(k * 16, 16)] = li
            pltpu.sync_copy(pay_hbm.at[pl.ds(base, 80)], rows_v)
            pltpu.sync_copy(rows_v, acc.at[lidx.at[0]], add=True)
            return carry

        lax.fori_loop(0, (_E // _NS) // 80, chunk, 0)
        plsc.subcore_barrier()

        span = _ACCR // _NS
        pltpu.sync_copy(acc.at[pl.ds(s * span, span)],
                        out_hbm.at[pl.ds(c * _ACCR + s * span, span)])

    return sk(payload, dst)


def _bspec(blk, w):
    return pl.BlockSpec((blk, w), lambda i: (i, 0))


def _const_spec(shape):
    return pl.BlockSpec(shape, lambda i: (0, 0))


def _bn_affine(stats, g, be, n):
    s = stats[0:1, :]
    q = stats[1:2, :]
    m = s / n
    v = q / n - m * m
    scale = g * lax.rsqrt(v + 1e-5)
    return scale, be - m * scale


def _acc_stats(ref, y):
    upd = jnp.concatenate(
        [jnp.sum(y, axis=0, keepdims=True),
         jnp.sum(y * y, axis=0, keepdims=True),
         jnp.zeros((6, _D), _f32)], axis=0)

    @pl.when(pl.program_id(0) == 0)
    def _():
        ref[...] = jnp.zeros_like(ref)

    ref[...] += upd


def _pack_table(x, pos, w_in, b_in):
    def body(x_ref, p_ref, w_ref, b_ref, o_ref):
        h = x_ref[...] * w_ref[...] + b_ref[...]
        o_ref[...] = jnp.concatenate(
            [h, p_ref[...], jnp.zeros((_NBLK, _W - _D - 3), _f32)], axis=1)

    return pl.pallas_call(
        body,
        grid=(_N // _NBLK,),
        in_specs=[_bspec(_NBLK, 1), _bspec(_NBLK, 3),
                  _const_spec((1, _D)), _const_spec((1, _D))],
        out_specs=_bspec(_NBLK, _W),
        out_shape=jax.ShapeDtypeStruct((_N, _W), _f32),
    )(x, pos, w_in, b_in)


def _edge_pass_b(G, w1at, w1bt, w1c, b1):
    def body(gd, gs, wa, wb, wc, b, y_ref, s_ref):
        hd = gd[:, :_D]
        hs = gs[:, :_D]
        pd = gd[:, _D:_D + 3] - gs[:, _D:_D + 3]
        dist = jnp.sqrt(jnp.sum(pd * pd, axis=1, keepdims=True))
        y1 = (jnp.dot(hd, wa[...], preferred_element_type=_f32)
              + jnp.dot(hs, wb[...], preferred_element_type=_f32)
              + dist * wc[...] + b[...])
        y_ref[...] = jnp.concatenate(
            [y1, pd, dist, jnp.zeros((_EBLK, _W - _D - 4), _f32)], axis=1)
        _acc_stats(s_ref, y1)

    nblk = _E // _EBLK
    return pl.pallas_call(
        body,
        grid=(nblk,),
        in_specs=[
            pl.BlockSpec((_EBLK, _W), lambda i: (i, 0)),
            pl.BlockSpec((_EBLK, _W), lambda i: (i + nblk, 0)),
            _const_spec((_D, _D)), _const_spec((_D, _D)),
            _const_spec((1, _D)), _const_spec((1, _D)),
        ],
        out_specs=[_bspec(_EBLK, _W), _const_spec((8, _D))],
        out_shape=[jax.ShapeDtypeStruct((_E, _W), _f32),
                   jax.ShapeDtypeStruct((8, _D), _f32)],
    )(G, G, w1at, w1bt, w1c, b1)


def _edge_pass_c(Y, s1, g1, be1, w2t, b2):
    def body(y_ref, s1_ref, g1_r, be1_r, w2_r, b2_r, s_ref):
        sc1, sh1 = _bn_affine(s1_ref[...], g1_r[...], be1_r[...], float(_E))
        z1 = jnp.maximum(y_ref[:, :_D] * sc1 + sh1, 0.0)
        y2 = jnp.dot(z1, w2_r[...], preferred_element_type=_f32) + b2_r[...]
        _acc_stats(s_ref, y2)

    return pl.pallas_call(
        body,
        grid=(_E // _EBLK,),
        in_specs=[_bspec(_EBLK, _W), _const_spec((8, _D)),
                  _const_spec((1, _D)), _const_spec((1, _D)),
                  _const_spec((_D, _D)), _const_spec((1, _D))],
        out_specs=_const_spec((8, _D)),
        out_shape=jax.ShapeDtypeStruct((8, _D), _f32),
    )(Y, s1, g1, be1, w2t, b2)


def _edge_pass_d(Y, s1, s2, g1, be1, w2t, b2, g2, be2, pw1t, pb1):
    def body(y_ref, s1_ref, s2_ref, g1_r, be1_r, w2_r, b2_r,
             g2_r, be2_r, pw1_r, pb1_r, s_ref):
        sc1, sh1 = _bn_affine(s1_ref[...], g1_r[...], be1_r[...], float(_E))
        z1 = jnp.maximum(y_ref[:, :_D] * sc1 + sh1, 0.0)
        y2 = jnp.dot(z1, w2_r[...], preferred_element_type=_f32) + b2_r[...]
        sc2, sh2 = _bn_affine(s2_ref[...], g2_r[...], be2_r[...], float(_E))
        z2 = jnp.maximum(y2 * sc2 + sh2, 0.0)
        y3 = jnp.dot(z2, pw1_r[...], preferred_element_type=_f32) + pb1_r[...]
        _acc_stats(s_ref, y3)

    return pl.pallas_call(
        body,
        grid=(_E // _EBLK,),
        in_specs=[_bspec(_EBLK, _W), _const_spec((8, _D)), _const_spec((8, _D)),
                  _const_spec((1, _D)), _const_spec((1, _D)),
                  _const_spec((_D, _D)), _const_spec((1, _D)),
                  _const_spec((1, _D)), _const_spec((1, _D)),
                  _const_spec((_D, _D)), _const_spec((1, _D))],
        out_specs=_const_spec((8, _D)),
        out_shape=jax.ShapeDtypeStruct((8, _D), _f32),
    )(Y, s1, s2, g1, be1, w2t, b2, g2, be2, pw1t, pb1)


def _edge_pass_e(Y, s1, s2, s3, g1, be1, w2t, b2, g2, be2,
                 pw1t, pb1, g3, be3, pw2t, pb2):
    def body(y_ref, s1_ref, s2_ref, s3_ref, g1_r, be1_r, w2_r, b2_r,
             g2_r, be2_r, pw1_r, pb1_r, g3_r, be3_r, pw2_r, pb2_r, o_ref):
        sc1, sh1 = _bn_affine(s1_ref[...], g1_r[...], be1_r[...], float(_E))
        z1 = jnp.maximum(y_ref[:, :_D] * sc1 + sh1, 0.0)
        y2 = jnp.dot(z1, w2_r[...], preferred_element_type=_f32) + b2_r[...]
        sc2, sh2 = _bn_affine(s2_ref[...], g2_r[...], be2_r[...], float(_E))
        z2 = jnp.maximum(y2 * sc2 + sh2, 0.0)
        y3 = jnp.dot(z2, pw1_r[...], preferred_element_type=_f32) + pb1_r[...]
        sc3, sh3 = _bn_affine(s3_ref[...], g3_r[...], be3_r[...], float(_E))
        z3 = jnp.maximum(y3 * sc3 + sh3, 0.0)
        pw = jnp.dot(z3, pw2_r[...], preferred_element_type=_f32) + pb2_r[...]
        wpos = y_ref[:, _D:_D + 3] * pw
        o_ref[...] = jnp.concatenate(
            [z2, wpos, jnp.ones((_EBLK, 1), _f32),
             jnp.zeros((_EBLK, _W - _D - 4), _f32)], axis=1)

    return pl.pallas_call(
        body,
        grid=(_E // _EBLK,),
        in_specs=[_bspec(_EBLK, _W),
                  _const_spec((8, _D)), _const_spec((8, _D)),
                  _const_spec((8, _D)),
                  _const_spec((1, _D)), _const_spec((1, _D)),
                  _const_spec((_D, _D)), _const_spec((1, _D)),
                  _const_spec((1, _D)), _const_spec((1, _D)),
                  _const_spec((_D, _D)), _const_spec((1, _D)),
                  _const_spec((1, _D)), _const_spec((1, _D)),
                  _const_spec((_D, 1)), _const_spec((1, 1))],
        out_specs=_bspec(_EBLK, _W),
        out_shape=jax.ShapeDtypeStruct((_E, _W), _f32),
    )(Y, s1, s2, s3, g1, be1, w2t, b2, g2, be2, pw1t, pb1, g3, be3, pw2t, pb2)


def _node_pass_1(T, A, u1at, u1bt, ub1):
    def body(t_ref, a_ref, wa, wb, b, s_ref):
        h = t_ref[:, :_D]
        denom = jnp.maximum(a_ref[:, _D + 3:_D + 4], 1.0)
        magg = a_ref[:, :_D] / denom
        u1 = (jnp.dot(h, wa[...], preferred_element_type=_f32)
              + jnp.dot(magg, wb[...], preferred_element_type=_f32) + b[...])
        _acc_stats(s_ref, u1)

    return pl.pallas_call(
        body,
        grid=(_N // _NBLK,),
        in_specs=[_bspec(_NBLK, _W), _bspec(_NBLK, _W),
                  _const_spec((_D, _D)), _const_spec((_D, _D)),
                  _const_spec((1, _D))],
        out_specs=_const_spec((8, _D)),
        out_shape=jax.ShapeDtypeStruct((8, _D), _f32),
    )(T, A, u1at, u1bt, ub1)


def _node_pass_2(T, A, s1, u1at, u1bt, ub1, g1, be1, u2t, ub2):
    def body(t_ref, a_ref, s1_ref, wa, wb, b, g1_r, be1_r, w2_r, b2_r, s_ref):
        h = t_ref[:, :_D]
        denom = jnp.maximum(a_ref[:, _D + 3:_D + 4], 1.0)
        magg = a_ref[:, :_D] / denom
        u1 = (jnp.dot(h, wa[...], preferred_element_type=_f32)
              + jnp.dot(magg, wb[...], preferred_element_type=_f32) + b[...])
        sc1, sh1 = _bn_affine(s1_ref[...], g1_r[...], be1_r[...], float(_N))
        z1 = jnp.maximum(u1 * sc1 + sh1, 0.0)
        u2 = jnp.dot(z1, w2_r[...], preferred_element_type=_f32) + b2_r[...]
        _acc_stats(s_ref, u2)

    return pl.pallas_call(
        body,
        grid=(_N // _NBLK,),
        in_specs=[_bspec(_NBLK, _W), _bspec(_NBLK, _W), _const_spec((8, _D)),
                  _const_spec((_D, _D)), _const_spec((_D, _D)),
                  _const_spec((1, _D)), _const_spec((1, _D)),
                  _const_spec((1, _D)), _const_spec((_D, _D)),
                  _const_spec((1, _D))],
        out_specs=_const_spec((8, _D)),
        out_shape=jax.ShapeDtypeStruct((8, _D), _f32),
    )(T, A, s1, u1at, u1bt, ub1, g1, be1, u2t, ub2)


def _node_pass_3(T, A, s1, s2, u1at, u1bt, ub1, g1, be1, u2t, ub2,
                 g2, be2, owt, ob):
    def body(t_ref, a_ref, s1_ref, s2_ref, wa, wb, b, g1_r, be1_r,
             w2_r, b2_r, g2_r, be2_r, ow_r, ob_r, o_ref):
        h = t_ref[:, :_D]
        denom = jnp.maximum(a_ref[:, _D + 3:_D + 4], 1.0)
        magg = a_ref[:, :_D] / denom
        paggr = a_ref[:, _D:_D + 3] / denom
        u1 = (jnp.dot(h, wa[...], preferred_element_type=_f32)
              + jnp.dot(magg, wb[...], preferred_element_type=_f32) + b[...])
        sc1, sh1 = _bn_affine(s1_ref[...], g1_r[...], be1_r[...], float(_N))
        z1 = jnp.maximum(u1 * sc1 + sh1, 0.0)
        u2 = jnp.dot(z1, w2_r[...], preferred_element_type=_f32) + b2_r[...]
        sc2, sh2 = _bn_affine(s2_ref[...], g2_r[...], be2_r[...], float(_N))
        z2 = jnp.maximum(u2 * sc2 + sh2, 0.0)
        h_out = jnp.dot(z2, ow_r[...], preferred_element_type=_f32) + ob_r[...]
        pos_out = t_ref[:, _D:_D + 3] + paggr
        o_ref[...] = jnp.concatenate(
            [h_out, pos_out, jnp.zeros((_NBLK, _W - _D - 3), _f32)], axis=1)

    return pl.pallas_call(
        body,
        grid=(_N // _NBLK,),
        in_specs=[_bspec(_NBLK, _W), _bspec(_NBLK, _W),
                  _const_spec((8, _D)), _const_spec((8, _D)),
                  _const_spec((_D, _D)), _const_spec((_D, _D)),
                  _const_spec((1, _D)), _const_spec((1, _D)),
                  _const_spec((1, _D)), _const_spec((_D, _D)),
                  _const_spec((1, _D)), _const_spec((1, _D)),
                  _const_spec((1, _D)), _const_spec((_D, _D)),
                  _const_spec((1, _D))],
        out_specs=_bspec(_NBLK, _W),
        out_shape=jax.ShapeDtypeStruct((_N, _W), _f32),
    )(T, A, s1, s2, u1at, u1bt, ub1, g1, be1, u2t, ub2, g2, be2, owt, ob)


def _col_sum(T):
    def body(t_ref, s_ref):
        _acc_stats(s_ref, t_ref[:, :_D])

    return pl.pallas_call(
        body,
        grid=(_N // _NBLK,),
        in_specs=[_bspec(_NBLK, _W)],
        out_specs=_const_spec((8, _D)),
        out_shape=jax.ShapeDtypeStruct((8, _D), _f32),
    )(T)


def _heads(hsum_r, hsum_l, wvr, bvr, wor, bor, wvl, bvl, wol, bol,
           frr, frb, ftr, ftb, flr, flb, ftl, ftlb):
    def body(hr_ref, hl_ref, wvr_r, bvr_r, wor_r, bor_r, wvl_r, bvl_r,
             wol_r, bol_r, frr_r, frb_r, ftr_r, ftb_r, flr_r, flb_r,
             ftl_r, ftlb_r, o_ref):
        hr = hr_ref[0:1, :] / float(_N)
        hl = hl_ref[0:1, :] / float(_N)
        ar = (jnp.dot(jnp.dot(hl, wvr_r[...], preferred_element_type=_f32)
                      + bvr_r[...], wor_r[...], preferred_element_type=_f32)
              + bor_r[...])
        al = (jnp.dot(jnp.dot(hr, wvl_r[...], preferred_element_type=_f32)
                      + bvl_r[...], wol_r[...], preferred_element_type=_f32)
              + bol_r[...])
        rr = jnp.dot(ar, frr_r[...], preferred_element_type=_f32) + frb_r[...]
        tr = jnp.dot(ar, ftr_r[...], preferred_element_type=_f32) + ftb_r[...]
        rl = jnp.dot(al, flr_r[...], preferred_element_type=_f32) + flb_r[...]
        tl = jnp.dot(al, ftl_r[...], preferred_element_type=_f32) + ftlb_r[...]
        z4 = jnp.zeros((1, 4), _f32)
        row0 = jnp.concatenate([rr, tr, z4], axis=1)
        row1 = jnp.concatenate([rl, tl, z4], axis=1)
        o_ref[...] = jnp.concatenate(
            [row0, row1, jnp.zeros((6, 16), _f32)], axis=0)

    specs = ([_const_spec((8, _D))] * 2
             + [_const_spec((_D, _D)), _const_spec((1, _D))] * 4
             + [_const_spec((_D, 9)), _const_spec((1, 9)),
                _const_spec((_D, 3)), _const_spec((1, 3))] * 2)
    return pl.pallas_call(
        body,
        grid=(1,),
        in_specs=specs,
        out_specs=_const_spec((8, 16)),
        out_shape=jax.ShapeDtypeStruct((8, 16), _f32),
    )(hsum_r, hsum_l, wvr, bvr, wor, bor, wvl, bvl, wol, bol,
      frr, frb, ftr, ftb, flr, flb, ftl, ftlb)


def _coords(pos, RT, row):
    def body(p_ref, rt_ref, o_ref):
        p = p_ref[...]
        cols = []
        for j in range(3):
            c = (p[:, 0:1] * rt_ref[row, 3 * j]
                 + p[:, 1:2] * rt_ref[row, 3 * j + 1]
                 + p[:, 2:3] * rt_ref[row, 3 * j + 2]
                 + rt_ref[row, 9 + j])
            cols.append(c)
        o_ref[...] = jnp.concatenate(cols, axis=1)

    return pl.pallas_call(
        body,
        grid=(_N // _NBLK,),
        in_specs=[_bspec(_NBLK, 3), _const_spec((8, 16))],
        out_specs=_bspec(_NBLK, 3),
        out_shape=jax.ShapeDtypeStruct((_N, 3), _f32),
    )(pos, RT)


def _prep_layer(p):
    r = lambda a: a.reshape(1, -1)
    return dict(
        w1at=p["msg_W1"][:, :_D].T, w1bt=p["msg_W1"][:, _D:2 * _D].T,
        w1c=r(p["msg_W1"][:, 2 * _D]), b1=r(p["msg_b1"]),
        g1=r(p["msg_g1"]), be1=r(p["msg_be1"]),
        w2t=p["msg_W2"].T, b2=r(p["msg_b2"]),
        g2=r(p["msg_g2"]), be2=r(p["msg_be2"]),
        pw1t=p["pos_W1"].T, pb1=r(p["pos_b1"]),
        g3=r(p["pos_g1"]), be3=r(p["pos_be1"]),
        pw2t=p["pos_W2"].T, pb2=p["pos_b2"].reshape(1, 1),
        u1at=p["upd_W1"][:, :_D].T, u1bt=p["upd_W1"][:, _D:].T,
        ub1=r(p["upd_b1"]), ug1=r(p["upd_g1"]), ube1=r(p["upd_be1"]),
        u2t=p["upd_W2"].T, ub2=r(p["upd_b2"]),
        ug2=r(p["upd_g2"]), ube2=r(p["upd_be2"]),
        owt=p["out_W"].T, ob=r(p["out_b"]),
    )


def _mpnn_layer(T, idx_g, dst, q):
    G = _sc_gather(T, idx_g)
    Y, s1 = _edge_pass_b(G, q["w1at"], q["w1bt"], q["w1c"], q["b1"])
    s2 = _edge_pass_c(Y, s1, q["g1"], q["be1"], q["w2t"], q["b2"])
    s3 = _edge_pass_d(Y, s1, s2, q["g1"], q["be1"], q["w2t"], q["b2"],
                      q["g2"], q["be2"], q["pw1t"], q["pb1"])
    S = _edge_pass_e(Y, s1, s2, s3, q["g1"], q["be1"], q["w2t"], q["b2"],
                     q["g2"], q["be2"], q["pw1t"], q["pb1"],
                     q["g3"], q["be3"], q["pw2t"], q["pb2"])
    A_raw = _sc_scatter(S, dst)
    A = jnp.concatenate(
        [A_raw[:_HALF], A_raw[_ACCR:_ACCR + _HALF]], axis=0)
    n1 = _node_pass_1(T, A, q["u1at"], q["u1bt"], q["ub1"])
    n2 = _node_pass_2(T, A, n1, q["u1at"], q["u1bt"], q["ub1"],
                      q["ug1"], q["ube1"], q["u2t"], q["ub2"])
    return _node_pass_3(T, A, n1, n2, q["u1at"], q["u1bt"], q["ub1"],
                        q["ug1"], q["ube1"], q["u2t"], q["ub2"],
                        q["ug2"], q["ube2"], q["owt"], q["ob"])


def kernel(receptor_x, receptor_pos, ligand_x, ligand_pos, params,
           receptor_edge_index, ligand_edge_index):
    p = params
    r = lambda a: a.reshape(1, -1)

    Tr = _pack_table(receptor_x, receptor_pos,
                     r(p["lin_in_rec_W"][:, 0]), r(p["lin_in_rec_b"]))
    Tl = _pack_table(ligand_x, ligand_pos,
                     r(p["lin_in_lig_W"][:, 0]), r(p["lin_in_lig_b"]))

    rdst = receptor_edge_index[1]
    ridx = jnp.concatenate([rdst, receptor_edge_index[0]])
    ldst = ligand_edge_index[1]
    lidx = jnp.concatenate([ldst, ligand_edge_index[0]])

    q1, q2 = _prep_layer(p["rec_l1"]), _prep_layer(p["rec_l2"])
    q3, q4 = _prep_layer(p["lig_l1"]), _prep_layer(p["lig_l2"])

    Tr = _mpnn_layer(Tr, ridx, rdst, q1)
    Tr = _mpnn_layer(Tr, ridx, rdst, q2)
    Tl = _mpnn_layer(Tl, lidx, ldst, q3)
    Tl = _mpnn_layer(Tl, lidx, ldst, q4)

    hsum_r = _col_sum(Tr)
    hsum_l = _col_sum(Tl)

    ra, la = p["rec_attn"], p["lig_attn"]
    RT = _heads(hsum_r, hsum_l,
                ra["Wv"].T, r(ra["bv"]), ra["Wo"].T, r(ra["bo"]),
                la["Wv"].T, r(la["bv"]), la["Wo"].T, r(la["bo"]),
                p["fc_r_rec_W"].T, r(p["fc_r_rec_b"]),
                p["fc_t_rec_W"].T, r(p["fc_t_rec_b"]),
                p["fc_r_lig_W"].T, r(p["fc_r_lig_b"]),
                p["fc_t_lig_W"].T, r(p["fc_t_lig_b"]))

    rec_coords = _coords(receptor_pos, RT, 0)
    lig_coords = _coords(ligand_pos, RT, 1)
    return (rec_coords, lig_coords)


# speedup vs baseline: 2.4425x; 2.4425x over previous
import functools

import jax
import jax.numpy as jnp
from jax import lax
from jax.experimental import pallas as pl
from jax.experimental.pallas import tpu as pltpu
from jax.experimental.pallas import tpu_sc as plsc

_N = 50000
_E = 800000
_D = 64
_W = 128
_WY = 80
_EBLK = 4000
_NBLK = 2000
_NC = 2
_NS = 16
_QTR = 12500
_ACCR = 12544
_ZROWS = 112
_GOPS = (2 * _E) // 128
_GPW = (_GOPS + 31) // 32

_f32 = jnp.float32


def _sc_gather(table, idx):
    mesh = plsc.VectorSubcoreMesh(core_axis_name="c", subcore_axis_name="s")

    @functools.partial(
        pl.kernel,
        out_type=jax.ShapeDtypeStruct((2 * _E, _W), _f32),
        mesh=mesh,
        scratch_types=[
            pltpu.VMEM((128,), jnp.int32),
            pltpu.VMEM((128, _W), _f32),
            pltpu.SemaphoreType.DMA,
        ],
    )
    def gk(table_hbm, idx_hbm, out_hbm, idx_v, rows_v, sem):
        wid = lax.axis_index("s") * _NC + lax.axis_index("c")

        def body(i, carry):
            oprow = wid * _GPW + i

            @pl.when(oprow < _GOPS)
            def _():
                base = oprow * 128
                pltpu.sync_copy(idx_hbm.at[pl.ds(base, 128)], idx_v)
                pltpu.async_copy(table_hbm.at[idx_v], rows_v, sem).wait()
                pltpu.sync_copy(rows_v, out_hbm.at[pl.ds(base, 128)])

            return carry

        lax.fori_loop(0, _GPW, body, 0)

    return gk(table, idx)


def _sc_scatter(payload, dst, nbase):
    mesh = plsc.VectorSubcoreMesh(core_axis_name="c", subcore_axis_name="s")

    @functools.partial(
        pl.kernel,
        out_type=jax.ShapeDtypeStruct((2 * _ACCR, _W), _f32),
        mesh=mesh,
        scratch_types=[
            pltpu.VMEM((80,), jnp.int32),
            pltpu.VMEM((8, 80), jnp.int32),
            pltpu.VMEM((80, _W), _f32),
            pltpu.VMEM((_ZROWS, _W), _f32),
            pltpu.VMEM_SHARED((_ACCR, _W), _f32),
        ],
    )
    def sk(pay_hbm, dst_hbm, out_hbm, dst_v, lidx, rows_v, zbuf, acc):
        c = lax.axis_index("c")
        s = lax.axis_index("s")
        lo = nbase + c * _QTR
        hi = lo + _QTR

        def zrow(rr, carry):
            for k in range(_W // 16):
                zbuf[rr, pl.ds(k * 16, 16)] = jnp.zeros((16,), _f32)
            return carry

        lax.fori_loop(0, _ZROWS, zrow, 0)
        for k in range(7):
            pltpu.sync_copy(
                zbuf, acc.at[pl.ds(s * (7 * _ZROWS) + k * _ZROWS, _ZROWS)])
        plsc.subcore_barrier()

        tbase = s * (_E // _NS)
        iota = lax.broadcasted_iota(jnp.int32, (16,), 0)

        def chunk(i, carry):
            base = tbase + i * 80
            pltpu.sync_copy(dst_hbm.at[pl.ds(base, 80)], dst_v)
            for k in range(5):
                v = dst_v[pl.ds(k * 16, 16)]
                m = (v >= lo) & (v < hi)
                li = jnp.where(m, v - lo, _QTR + (iota & 7))
                lidx[0, pl.ds(k * 16, 16)] = li
            pltpu.sync_copy(pay_hbm.at[pl.ds(base, 80)], rows_v)
            pltpu.sync_copy(rows_v, acc.at[lidx.at[0]], add=True)
            return carry

        lax.fori_loop(0, (_E // _NS) // 80, chunk, 0)
        plsc.subcore_barrier()

        span = _ACCR // _NS
        pltpu.sync_copy(acc.at[pl.ds(s * span, span)],
                        out_hbm.at[pl.ds(c * _ACCR + s * span, span)])

    return sk(payload, dst)


def _bspec(blk, w):
    return pl.BlockSpec((blk, w), lambda i: (i, 0))


def _const_spec(shape):
    return pl.BlockSpec(shape, lambda i: (0, 0))


def _bn_affine(stats, g, be, n):
    s = stats[0:1, :]
    q = stats[1:2, :]
    m = s / n
    v = q / n - m * m
    scale = g * lax.rsqrt(v + 1e-5)
    return scale, be - m * scale


def _acc_stats(ref, y):
    upd = jnp.concatenate(
        [jnp.sum(y, axis=0, keepdims=True),
         jnp.sum(y * y, axis=0, keepdims=True),
         jnp.zeros((6, _D), _f32)], axis=0)

    @pl.when(pl.program_id(0) == 0)
    def _():
        ref[...] = jnp.zeros_like(ref)

    ref[...] += upd


def _pack_table(x, pos, w_in, b_in):
    def body(x_ref, p_ref, w_ref, b_ref, o_ref):
        h = x_ref[...] * w_ref[...] + b_ref[...]
        o_ref[...] = jnp.concatenate(
            [h, p_ref[...], jnp.zeros((_NBLK, _W - _D - 3), _f32)], axis=1)

    return pl.pallas_call(
        body,
        grid=(_N // _NBLK,),
        in_specs=[_bspec(_NBLK, 1), _bspec(_NBLK, 3),
                  _const_spec((1, _D)), _const_spec((1, _D))],
        out_specs=_bspec(_NBLK, _W),
        out_shape=jax.ShapeDtypeStruct((_N, _W), _f32),
    )(x, pos, w_in, b_in)


def _edge_pass_b(G, w1at, w1bt, w1c, b1):
    def body(gd, gs, wa, wb, wc, b, y_ref, s_ref):
        hd = gd[:, :_D]
        hs = gs[:, :_D]
        pd = gd[:, _D:_D + 3] - gs[:, _D:_D + 3]
        dist = jnp.sqrt(jnp.sum(pd * pd, axis=1, keepdims=True))
        y1 = (jnp.dot(hd, wa[...], preferred_element_type=_f32)
              + jnp.dot(hs, wb[...], preferred_element_type=_f32)
              + dist * wc[...] + b[...])
        y_ref[...] = jnp.concatenate(
            [y1, pd, dist, jnp.zeros((_EBLK, _WY - _D - 4), _f32)], axis=1)
        _acc_stats(s_ref, y1)

    nblk = _E // _EBLK
    return pl.pallas_call(
        body,
        grid=(nblk,),
        in_specs=[
            pl.BlockSpec((_EBLK, _W), lambda i: (i, 0)),
            pl.BlockSpec((_EBLK, _W), lambda i: (i + nblk, 0)),
            _const_spec((_D, _D)), _const_spec((_D, _D)),
            _const_spec((1, _D)), _const_spec((1, _D)),
        ],
        out_specs=[_bspec(_EBLK, _WY), _const_spec((8, _D))],
        out_shape=[jax.ShapeDtypeStruct((_E, _WY), _f32),
                   jax.ShapeDtypeStruct((8, _D), _f32)],
    )(G, G, w1at, w1bt, w1c, b1)


def _edge_pass_c(Y, s1, g1, be1, w2t, b2):
    def body(y_ref, s1_ref, g1_r, be1_r, w2_r, b2_r, s_ref):
        sc1, sh1 = _bn_affine(s1_ref[...], g1_r[...], be1_r[...], float(_E))
        z1 = jnp.maximum(y_ref[:, :_D] * sc1 + sh1, 0.0)
        y2 = jnp.dot(z1, w2_r[...], preferred_element_type=_f32) + b2_r[...]
        _acc_stats(s_ref, y2)

    return pl.pallas_call(
        body,
        grid=(_E // _EBLK,),
        in_specs=[_bspec(_EBLK, _WY), _const_spec((8, _D)),
                  _const_spec((1, _D)), _const_spec((1, _D)),
                  _const_spec((_D, _D)), _const_spec((1, _D))],
        out_specs=_const_spec((8, _D)),
        out_shape=jax.ShapeDtypeStruct((8, _D), _f32),
    )(Y, s1, g1, be1, w2t, b2)


def _edge_pass_d(Y, s1, s2, g1, be1, w2t, b2, g2, be2, pw1t, pb1):
    def body(y_ref, s1_ref, s2_ref, g1_r, be1_r, w2_r, b2_r,
             g2_r, be2_r, pw1_r, pb1_r, s_ref):
        sc1, sh1 = _bn_affine(s1_ref[...], g1_r[...], be1_r[...], float(_E))
        z1 = jnp.maximum(y_ref[:, :_D] * sc1 + sh1, 0.0)
        y2 = jnp.dot(z1, w2_r[...], preferred_element_type=_f32) + b2_r[...]
        sc2, sh2 = _bn_affine(s2_ref[...], g2_r[...], be2_r[...], float(_E))
        z2 = jnp.maximum(y2 * sc2 + sh2, 0.0)
        y3 = jnp.dot(z2, pw1_r[...], preferred_element_type=_f32) + pb1_r[...]
        _acc_stats(s_ref, y3)

    return pl.pallas_call(
        body,
        grid=(_E // _EBLK,),
        in_specs=[_bspec(_EBLK, _WY), _const_spec((8, _D)), _const_spec((8, _D)),
                  _const_spec((1, _D)), _const_spec((1, _D)),
                  _const_spec((_D, _D)), _const_spec((1, _D)),
                  _const_spec((1, _D)), _const_spec((1, _D)),
                  _const_spec((_D, _D)), _const_spec((1, _D))],
        out_specs=_const_spec((8, _D)),
        out_shape=jax.ShapeDtypeStruct((8, _D), _f32),
    )(Y, s1, s2, g1, be1, w2t, b2, g2, be2, pw1t, pb1)


def _edge_pass_e(Y, s1, s2, s3, g1, be1, w2t, b2, g2, be2,
                 pw1t, pb1, g3, be3, pw2t, pb2):
    def body(y_ref, s1_ref, s2_ref, s3_ref, g1_r, be1_r, w2_r, b2_r,
             g2_r, be2_r, pw1_r, pb1_r, g3_r, be3_r, pw2_r, pb2_r, o_ref):
        sc1, sh1 = _bn_affine(s1_ref[...], g1_r[...], be1_r[...], float(_E))
        z1 = jnp.maximum(y_ref[:, :_D] * sc1 + sh1, 0.0)
        y2 = jnp.dot(z1, w2_r[...], preferred_element_type=_f32) + b2_r[...]
        sc2, sh2 = _bn_affine(s2_ref[...], g2_r[...], be2_r[...], float(_E))
        z2 = jnp.maximum(y2 * sc2 + sh2, 0.0)
        y3 = jnp.dot(z2, pw1_r[...], preferred_element_type=_f32) + pb1_r[...]
        sc3, sh3 = _bn_affine(s3_ref[...], g3_r[...], be3_r[...], float(_E))
        z3 = jnp.maximum(y3 * sc3 + sh3, 0.0)
        pw = jnp.dot(z3, pw2_r[...], preferred_element_type=_f32) + pb2_r[...]
        wpos = y_ref[:, _D:_D + 3] * pw
        o_ref[...] = jnp.concatenate(
            [z2, wpos, jnp.ones((_EBLK, 1), _f32),
             jnp.zeros((_EBLK, _W - _D - 4), _f32)], axis=1)

    return pl.pallas_call(
        body,
        grid=(_E // _EBLK,),
        in_specs=[_bspec(_EBLK, _WY),
                  _const_spec((8, _D)), _const_spec((8, _D)),
                  _const_spec((8, _D)),
                  _const_spec((1, _D)), _const_spec((1, _D)),
                  _const_spec((_D, _D)), _const_spec((1, _D)),
                  _const_spec((1, _D)), _const_spec((1, _D)),
                  _const_spec((_D, _D)), _const_spec((1, _D)),
                  _const_spec((1, _D)), _const_spec((1, _D)),
                  _const_spec((_D, 1)), _const_spec((1, 1))],
        out_specs=_bspec(_EBLK, _W),
        out_shape=jax.ShapeDtypeStruct((_E, _W), _f32),
    )(Y, s1, s2, s3, g1, be1, w2t, b2, g2, be2, pw1t, pb1, g3, be3, pw2t, pb2)


def _node_pass_1(T, A, u1at, u1bt, ub1):
    def body(t_ref, a_ref, wa, wb, b, s_ref):
        h = t_ref[:, :_D]
        denom = jnp.maximum(a_ref[:, _D + 3:_D + 4], 1.0)
        magg = a_ref[:, :_D] / denom
        u1 = (jnp.dot(h, wa[...], preferred_element_type=_f32)
              + jnp.dot(magg, wb[...], preferred_element_type=_f32) + b[...])
        _acc_stats(s_ref, u1)

    return pl.pallas_call(
        body,
        grid=(_N // _NBLK,),
        in_specs=[_bspec(_NBLK, _W), _bspec(_NBLK, _W),
                  _const_spec((_D, _D)), _const_spec((_D, _D)),
                  _const_spec((1, _D))],
        out_specs=_const_spec((8, _D)),
        out_shape=jax.ShapeDtypeStruct((8, _D), _f32),
    )(T, A, u1at, u1bt, ub1)


def _node_pass_2(T, A, s1, u1at, u1bt, ub1, g1, be1, u2t, ub2):
    def body(t_ref, a_ref, s1_ref, wa, wb, b, g1_r, be1_r, w2_r, b2_r, s_ref):
        h = t_ref[:, :_D]
        denom = jnp.maximum(a_ref[:, _D + 3:_D + 4], 1.0)
        magg = a_ref[:, :_D] / denom
        u1 = (jnp.dot(h, wa[...], preferred_element_type=_f32)
              + jnp.dot(magg, wb[...], preferred_element_type=_f32) + b[...])
        sc1, sh1 = _bn_affine(s1_ref[...], g1_r[...], be1_r[...], float(_N))
        z1 = jnp.maximum(u1 * sc1 + sh1, 0.0)
        u2 = jnp.dot(z1, w2_r[...], preferred_element_type=_f32) + b2_r[...]
        _acc_stats(s_ref, u2)

    return pl.pallas_call(
        body,
        grid=(_N // _NBLK,),
        in_specs=[_bspec(_NBLK, _W), _bspec(_NBLK, _W), _const_spec((8, _D)),
                  _const_spec((_D, _D)), _const_spec((_D, _D)),
                  _const_spec((1, _D)), _const_spec((1, _D)),
                  _const_spec((1, _D)), _const_spec((_D, _D)),
                  _const_spec((1, _D))],
        out_specs=_const_spec((8, _D)),
        out_shape=jax.ShapeDtypeStruct((8, _D), _f32),
    )(T, A, s1, u1at, u1bt, ub1, g1, be1, u2t, ub2)


def _node_pass_3(T, A, s1, s2, u1at, u1bt, ub1, g1, be1, u2t, ub2,
                 g2, be2, owt, ob):
    def body(t_ref, a_ref, s1_ref, s2_ref, wa, wb, b, g1_r, be1_r,
             w2_r, b2_r, g2_r, be2_r, ow_r, ob_r, o_ref):
        h = t_ref[:, :_D]
        denom = jnp.maximum(a_ref[:, _D + 3:_D + 4], 1.0)
        magg = a_ref[:, :_D] / denom
        paggr = a_ref[:, _D:_D + 3] / denom
        u1 = (jnp.dot(h, wa[...], preferred_element_type=_f32)
              + jnp.dot(magg, wb[...], preferred_element_type=_f32) + b[...])
        sc1, sh1 = _bn_affine(s1_ref[...], g1_r[...], be1_r[...], float(_N))
        z1 = jnp.maximum(u1 * sc1 + sh1, 0.0)
        u2 = jnp.dot(z1, w2_r[...], preferred_element_type=_f32) + b2_r[...]
        sc2, sh2 = _bn_affine(s2_ref[...], g2_r[...], be2_r[...], float(_N))
        z2 = jnp.maximum(u2 * sc2 + sh2, 0.0)
        h_out = jnp.dot(z2, ow_r[...], preferred_element_type=_f32) + ob_r[...]
        pos_out = t_ref[:, _D:_D + 3] + paggr
        o_ref[...] = jnp.concatenate(
            [h_out, pos_out, jnp.zeros((_NBLK, _W - _D - 3), _f32)], axis=1)

    return pl.pallas_call(
        body,
        grid=(_N // _NBLK,),
        in_specs=[_bspec(_NBLK, _W), _bspec(_NBLK, _W),
                  _const_spec((8, _D)), _const_spec((8, _D)),
                  _const_spec((_D, _D)), _const_spec((_D, _D)),
                  _const_spec((1, _D)), _const_spec((1, _D)),
                  _const_spec((1, _D)), _const_spec((_D, _D)),
                  _const_spec((1, _D)), _const_spec((1, _D)),
                  _const_spec((1, _D)), _const_spec((_D, _D)),
                  _const_spec((1, _D))],
        out_specs=_bspec(_NBLK, _W),
        out_shape=jax.ShapeDtypeStruct((_N, _W), _f32),
    )(T, A, s1, s2, u1at, u1bt, ub1, g1, be1, u2t, ub2, g2, be2, owt, ob)


def _col_sum(T):
    def body(t_ref, s_ref):
        _acc_stats(s_ref, t_ref[:, :_D])

    return pl.pallas_call(
        body,
        grid=(_N // _NBLK,),
        in_specs=[_bspec(_NBLK, _W)],
        out_specs=_const_spec((8, _D)),
        out_shape=jax.ShapeDtypeStruct((8, _D), _f32),
    )(T)


def _heads(hsum_r, hsum_l, wvr, bvr, wor, bor, wvl, bvl, wol, bol,
           frr, frb, ftr, ftb, flr, flb, ftl, ftlb):
    def body(hr_ref, hl_ref, wvr_r, bvr_r, wor_r, bor_r, wvl_r, bvl_r,
             wol_r, bol_r, frr_r, frb_r, ftr_r, ftb_r, flr_r, flb_r,
             ftl_r, ftlb_r, o_ref):
        hr = hr_ref[0:1, :] / float(_N)
        hl = hl_ref[0:1, :] / float(_N)
        ar = (jnp.dot(jnp.dot(hl, wvr_r[...], preferred_element_type=_f32)
                      + bvr_r[...], wor_r[...], preferred_element_type=_f32)
              + bor_r[...])
        al = (jnp.dot(jnp.dot(hr, wvl_r[...], preferred_element_type=_f32)
                      + bvl_r[...], wol_r[...], preferred_element_type=_f32)
              + bol_r[...])
        rr = jnp.dot(ar, frr_r[...], preferred_element_type=_f32) + frb_r[...]
        tr = jnp.dot(ar, ftr_r[...], preferred_element_type=_f32) + ftb_r[...]
        rl = jnp.dot(al, flr_r[...], preferred_element_type=_f32) + flb_r[...]
        tl = jnp.dot(al, ftl_r[...], preferred_element_type=_f32) + ftlb_r[...]
        z4 = jnp.zeros((1, 4), _f32)
        row0 = jnp.concatenate([rr, tr, z4], axis=1)
        row1 = jnp.concatenate([rl, tl, z4], axis=1)
        o_ref[...] = jnp.concatenate(
            [row0, row1, jnp.zeros((6, 16), _f32)], axis=0)

    specs = ([_const_spec((8, _D))] * 2
             + [_const_spec((_D, _D)), _const_spec((1, _D))] * 4
             + [_const_spec((_D, 9)), _const_spec((1, 9)),
                _const_spec((_D, 3)), _const_spec((1, 3))] * 2)
    return pl.pallas_call(
        body,
        grid=(1,),
        in_specs=specs,
        out_specs=_const_spec((8, 16)),
        out_shape=jax.ShapeDtypeStruct((8, 16), _f32),
    )(hsum_r, hsum_l, wvr, bvr, wor, bor, wvl, bvl, wol, bol,
      frr, frb, ftr, ftb, flr, flb, ftl, ftlb)


def _coords(pos, RT, row):
    def body(p_ref, rt_ref, o_ref):
        p = p_ref[...]
        cols = []
        for j in range(3):
            c = (p[:, 0:1] * rt_ref[row, 3 * j]
                 + p[:, 1:2] * rt_ref[row, 3 * j + 1]
                 + p[:, 2:3] * rt_ref[row, 3 * j + 2]
                 + rt_ref[row, 9 + j])
            cols.append(c)
        o_ref[...] = jnp.concatenate(cols, axis=1)

    return pl.pallas_call(
        body,
        grid=(_N // _NBLK,),
        in_specs=[_bspec(_NBLK, 3), _const_spec((8, 16))],
        out_specs=_bspec(_NBLK, 3),
        out_shape=jax.ShapeDtypeStruct((_N, 3), _f32),
    )(pos, RT)


def _prep_layer(p):
    r = lambda a: a.reshape(1, -1)
    return dict(
        w1at=p["msg_W1"][:, :_D].T, w1bt=p["msg_W1"][:, _D:2 * _D].T,
        w1c=r(p["msg_W1"][:, 2 * _D]), b1=r(p["msg_b1"]),
        g1=r(p["msg_g1"]), be1=r(p["msg_be1"]),
        w2t=p["msg_W2"].T, b2=r(p["msg_b2"]),
        g2=r(p["msg_g2"]), be2=r(p["msg_be2"]),
        pw1t=p["pos_W1"].T, pb1=r(p["pos_b1"]),
        g3=r(p["pos_g1"]), be3=r(p["pos_be1"]),
        pw2t=p["pos_W2"].T, pb2=p["pos_b2"].reshape(1, 1),
        u1at=p["upd_W1"][:, :_D].T, u1bt=p["upd_W1"][:, _D:].T,
        ub1=r(p["upd_b1"]), ug1=r(p["upd_g1"]), ube1=r(p["upd_be1"]),
        u2t=p["upd_W2"].T, ub2=r(p["upd_b2"]),
        ug2=r(p["upd_g2"]), ube2=r(p["upd_be2"]),
        owt=p["out_W"].T, ob=r(p["out_b"]),
    )


def _mpnn_layer(T, idx_g, dst, q):
    G = _sc_gather(T, idx_g)
    Y, s1 = _edge_pass_b(G, q["w1at"], q["w1bt"], q["w1c"], q["b1"])
    s2 = _edge_pass_c(Y, s1, q["g1"], q["be1"], q["w2t"], q["b2"])
    s3 = _edge_pass_d(Y, s1, s2, q["g1"], q["be1"], q["w2t"], q["b2"],
                      q["g2"], q["be2"], q["pw1t"], q["pb1"])
    S = _edge_pass_e(Y, s1, s2, s3, q["g1"], q["be1"], q["w2t"], q["b2"],
                     q["g2"], q["be2"], q["pw1t"], q["pb1"],
                     q["g3"], q["be3"], q["pw2t"], q["pb2"])
    A0 = _sc_scatter(S, dst, 0)
    A1 = _sc_scatter(S, dst, 2 * _QTR)
    A = jnp.concatenate(
        [A0[:_QTR], A0[_ACCR:_ACCR + _QTR],
         A1[:_QTR], A1[_ACCR:_ACCR + _QTR]], axis=0)
    n1 = _node_pass_1(T, A, q["u1at"], q["u1bt"], q["ub1"])
    n2 = _node_pass_2(T, A, n1, q["u1at"], q["u1bt"], q["ub1"],
                      q["ug1"], q["ube1"], q["u2t"], q["ub2"])
    return _node_pass_3(T, A, n1, n2, q["u1at"], q["u1bt"], q["ub1"],
                        q["ug1"], q["ube1"], q["u2t"], q["ub2"],
                        q["ug2"], q["ube2"], q["owt"], q["ob"])


def kernel(receptor_x, receptor_pos, ligand_x, ligand_pos, params,
           receptor_edge_index, ligand_edge_index):
    p = params
    r = lambda a: a.reshape(1, -1)

    Tr = _pack_table(receptor_x, receptor_pos,
                     r(p["lin_in_rec_W"][:, 0]), r(p["lin_in_rec_b"]))
    Tl = _pack_table(ligand_x, ligand_pos,
                     r(p["lin_in_lig_W"][:, 0]), r(p["lin_in_lig_b"]))

    rdst = receptor_edge_index[1]
    ridx = jnp.concatenate([rdst, receptor_edge_index[0]])
    ldst = ligand_edge_index[1]
    lidx = jnp.concatenate([ldst, ligand_edge_index[0]])

    q1, q2 = _prep_layer(p["rec_l1"]), _prep_layer(p["rec_l2"])
    q3, q4 = _prep_layer(p["lig_l1"]), _prep_layer(p["lig_l2"])

    Tr = _mpnn_layer(Tr, ridx, rdst, q1)
    Tr = _mpnn_layer(Tr, ridx, rdst, q2)
    Tl = _mpnn_layer(Tl, lidx, ldst, q3)
    Tl = _mpnn_layer(Tl, lidx, ldst, q4)

    hsum_r = _col_sum(Tr)
    hsum_l = _col_sum(Tl)

    ra, la = p["rec_attn"], p["lig_attn"]
    RT = _heads(hsum_r, hsum_l,
                ra["Wv"].T, r(ra["bv"]), ra["Wo"].T, r(ra["bo"]),
                la["Wv"].T, r(la["bv"]), la["Wo"].T, r(la["bo"]),
                p["fc_r_rec_W"].T, r(p["fc_r_rec_b"]),
                p["fc_t_rec_W"].T, r(p["fc_t_rec_b"]),
                p["fc_r_lig_W"].T, r(p["fc_r_lig_b"]),
                p["fc_t_lig_W"].T, r(p["fc_t_lig_b"]))

    rec_coords = _coords(receptor_pos, RT, 0)
    lig_coords = _coords(ligand_pos, RT, 1)
    return (rec_coords, lig_coords)


# speedup vs baseline: 3.7528x; 1.5365x over previous
import functools

import jax
import jax.numpy as jnp
from jax import lax
from jax.experimental import pallas as pl
from jax.experimental.pallas import tpu as pltpu
from jax.experimental.pallas import tpu_sc as plsc

_N = 50000
_E = 800000
_D = 64
_W = 128
_WY = 80
_EBLK = 4000
_NBLK = 2000
_NC = 2
_NS = 16
_QTR = 12500
_ACCR = 12544
_ZROWS = 56
_GOPS = (2 * _E) // 128
_GPW = 392

_f32 = jnp.float32


def _sc_gather(table, idx):
    mesh = plsc.VectorSubcoreMesh(core_axis_name="c", subcore_axis_name="s")

    @functools.partial(
        pl.kernel,
        out_type=jax.ShapeDtypeStruct((2 * _E, _W), _f32),
        mesh=mesh,
        scratch_types=[
            pltpu.VMEM((128,), jnp.int32),
            pltpu.VMEM((128,), jnp.int32),
            pltpu.VMEM((128, _W), _f32),
            pltpu.VMEM((128, _W), _f32),
            pltpu.SemaphoreType.DMA,
            pltpu.SemaphoreType.DMA,
        ],
    )
    def gk(table_hbm, idx_hbm, out_hbm, i0, i1, r0, r1, g0, g1):
        wid = lax.axis_index("s") * _NC + lax.axis_index("c")
        ib = (i0, i1)
        rb = (r0, r1)
        gs = (g0, g1)
        base = wid * _GPW
        limit = jnp.minimum(base + _GPW, _GOPS)

        def start(j, b):
            @pl.when(j < limit)
            def _():
                pltpu.sync_copy(idx_hbm.at[pl.ds(j * 128, 128)], ib[b])
                pltpu.async_copy(table_hbm.at[ib[b]], rb[b], gs[b])

        def finish(j, b):
            @pl.when(j < limit)
            def _():
                pltpu.make_async_copy(table_hbm.at[ib[b]], rb[b], gs[b]).wait()
                pltpu.sync_copy(rb[b], out_hbm.at[pl.ds(j * 128, 128)])
        start(base, 0)
        start(base + 1, 1)

        def body(g, carry):
            j = base + 2 * g
            finish(j, 0)
            start(j + 2, 0)
            finish(j + 1, 1)
            start(j + 3, 1)
            return carry

        lax.fori_loop(0, _GPW // 2, body, 0)

    return gk(table, idx)


def _sc_scatter(payload, dst, nbase):
    mesh = plsc.VectorSubcoreMesh(core_axis_name="c", subcore_axis_name="s")

    @functools.partial(
        pl.kernel,
        out_type=jax.ShapeDtypeStruct((2 * _ACCR, _W), _f32),
        mesh=mesh,
        scratch_types=[
            pltpu.VMEM((80,), jnp.int32),
            pltpu.VMEM((80,), jnp.int32),
            pltpu.VMEM((8, 80), jnp.int32),
            pltpu.VMEM((8, 80), jnp.int32),
            pltpu.VMEM((80, _W), _f32),
            pltpu.VMEM((80, _W), _f32),
            pltpu.VMEM((_ZROWS, _W), _f32),
            pltpu.VMEM_SHARED((_ACCR, _W), _f32),
            pltpu.SemaphoreType.DMA,
            pltpu.SemaphoreType.DMA,
            pltpu.SemaphoreType.DMA,
            pltpu.SemaphoreType.DMA,
        ],
    )
    def sk(pay_hbm, dst_hbm, out_hbm, d0, d1, l0, l1, r0, r1, zbuf, acc,
           sd0, sd1, sp0, sp1):
        c = lax.axis_index("c")
        s = lax.axis_index("s")
        lo = nbase + c * _QTR
        hi = lo + _QTR
        db = (d0, d1)
        lb = (l0, l1)
        rb = (r0, r1)
        sd = (sd0, sd1)
        sp = (sp0, sp1)

        def zrow(rr, carry):
            for k in range(_W // 16):
                zbuf[rr, pl.ds(k * 16, 16)] = jnp.zeros((16,), _f32)
            return carry

        lax.fori_loop(0, _ZROWS, zrow, 0)
        for k in range(_ACCR // _NS // _ZROWS):
            pltpu.sync_copy(
                zbuf,
                acc.at[pl.ds(s * (_ACCR // _NS) + k * _ZROWS, _ZROWS)])
        plsc.subcore_barrier()

        nchunk = (_E // _NS) // 80
        tbase = s * (_E // _NS)
        iota = lax.broadcasted_iota(jnp.int32, (16,), 0)

        def start(i, b):
            @pl.when(i < nchunk)
            def _():
                base = tbase + i * 80
                pltpu.async_copy(dst_hbm.at[pl.ds(base, 80)], db[b], sd[b])
                pltpu.async_copy(pay_hbm.at[pl.ds(base, 80)], rb[b], sp[b])

        def finish(i, b):
            @pl.when(i < nchunk)
            def _():
                base = tbase + i * 80
                pltpu.make_async_copy(
                    dst_hbm.at[pl.ds(base, 80)], db[b], sd[b]).wait()
                for k in range(5):
                    v = db[b][pl.ds(k * 16, 16)]
                    m = (v >= lo) & (v < hi)
                    li = jnp.where(m, v - lo, _QTR + (iota & 7))
                    lb[b][0, pl.ds(k * 16, 16)] = li
                pltpu.make_async_copy(
                    pay_hbm.at[pl.ds(base, 80)], rb[b], sp[b]).wait()
                pltpu.sync_copy(rb[b], acc.at[lb[b].at[0]], add=True)

        start(0, 0)
        start(1, 1)

        def pair(g, carry):
            i = 2 * g
            finish(i, 0)
            start(i + 2, 0)
            finish(i + 1, 1)
            start(i + 3, 1)
            return carry

        lax.fori_loop(0, (nchunk + 1) // 2, pair, 0)
        plsc.subcore_barrier()

        span = _ACCR // _NS
        pltpu.sync_copy(acc.at[pl.ds(s * span, span)],
                        out_hbm.at[pl.ds(c * _ACCR + s * span, span)])

    return sk(payload, dst)


def _bspec(blk, w):
    return pl.BlockSpec((blk, w), lambda i: (i, 0))


def _const_spec(shape):
    return pl.BlockSpec(shape, lambda i: (0, 0))


def _bn_affine(stats, g, be, n):
    s = stats[0:1, :]
    q = stats[1:2, :]
    m = s / n
    v = q / n - m * m
    scale = g * lax.rsqrt(v + 1e-5)
    return scale, be - m * scale


def _acc_stats(ref, y):
    upd = jnp.concatenate(
        [jnp.sum(y, axis=0, keepdims=True),
         jnp.sum(y * y, axis=0, keepdims=True),
         jnp.zeros((6, _D), _f32)], axis=0)

    @pl.when(pl.program_id(0) == 0)
    def _():
        ref[...] = jnp.zeros_like(ref)

    ref[...] += upd


def _pack_table(x, pos, w_in, b_in):
    def body(x_ref, p_ref, w_ref, b_ref, o_ref):
        h = x_ref[...] * w_ref[...] + b_ref[...]
        o_ref[...] = jnp.concatenate(
            [h, p_ref[...], jnp.zeros((_NBLK, _W - _D - 3), _f32)], axis=1)

    return pl.pallas_call(
        body,
        grid=(_N // _NBLK,),
        in_specs=[_bspec(_NBLK, 1), _bspec(_NBLK, 3),
                  _const_spec((1, _D)), _const_spec((1, _D))],
        out_specs=_bspec(_NBLK, _W),
        out_shape=jax.ShapeDtypeStruct((_N, _W), _f32),
    )(x, pos, w_in, b_in)


def _edge_pass_b(G, w1at, w1bt, w1c, b1):
    def body(gd, gs, wa, wb, wc, b, y_ref, s_ref):
        hd = gd[:, :_D]
        hs = gs[:, :_D]
        pd = gd[:, _D:_D + 3] - gs[:, _D:_D + 3]
        dist = jnp.sqrt(jnp.sum(pd * pd, axis=1, keepdims=True))
        y1 = (jnp.dot(hd, wa[...], preferred_element_type=_f32)
              + jnp.dot(hs, wb[...], preferred_element_type=_f32)
              + dist * wc[...] + b[...])
        y_ref[...] = jnp.concatenate(
            [y1, pd, dist, jnp.zeros((_EBLK, _WY - _D - 4), _f32)], axis=1)
        _acc_stats(s_ref, y1)

    nblk = _E // _EBLK
    return pl.pallas_call(
        body,
        grid=(nblk,),
        in_specs=[
            pl.BlockSpec((_EBLK, _W), lambda i: (i, 0)),
            pl.BlockSpec((_EBLK, _W), lambda i: (i + nblk, 0)),
            _const_spec((_D, _D)), _const_spec((_D, _D)),
            _const_spec((1, _D)), _const_spec((1, _D)),
        ],
        out_specs=[_bspec(_EBLK, _WY), _const_spec((8, _D))],
        out_shape=[jax.ShapeDtypeStruct((_E, _WY), _f32),
                   jax.ShapeDtypeStruct((8, _D), _f32)],
    )(G, G, w1at, w1bt, w1c, b1)


def _edge_pass_c(Y, s1, g1, be1, w2t, b2):
    def body(y_ref, s1_ref, g1_r, be1_r, w2_r, b2_r, s_ref):
        sc1, sh1 = _bn_affine(s1_ref[...], g1_r[...], be1_r[...], float(_E))
        z1 = jnp.maximum(y_ref[:, :_D] * sc1 + sh1, 0.0)
        y2 = jnp.dot(z1, w2_r[...], preferred_element_type=_f32) + b2_r[...]
        _acc_stats(s_ref, y2)

    return pl.pallas_call(
        body,
        grid=(_E // _EBLK,),
        in_specs=[_bspec(_EBLK, _WY), _const_spec((8, _D)),
                  _const_spec((1, _D)), _const_spec((1, _D)),
                  _const_spec((_D, _D)), _const_spec((1, _D))],
        out_specs=_const_spec((8, _D)),
        out_shape=jax.ShapeDtypeStruct((8, _D), _f32),
    )(Y, s1, g1, be1, w2t, b2)


def _edge_pass_d(Y, s1, s2, g1, be1, w2t, b2, g2, be2, pw1t, pb1):
    def body(y_ref, s1_ref, s2_ref, g1_r, be1_r, w2_r, b2_r,
             g2_r, be2_r, pw1_r, pb1_r, s_ref):
        sc1, sh1 = _bn_affine(s1_ref[...], g1_r[...], be1_r[...], float(_E))
        z1 = jnp.maximum(y_ref[:, :_D] * sc1 + sh1, 0.0)
        y2 = jnp.dot(z1, w2_r[...], preferred_element_type=_f32) + b2_r[...]
        sc2, sh2 = _bn_affine(s2_ref[...], g2_r[...], be2_r[...], float(_E))
        z2 = jnp.maximum(y2 * sc2 + sh2, 0.0)
        y3 = jnp.dot(z2, pw1_r[...], preferred_element_type=_f32) + pb1_r[...]
        _acc_stats(s_ref, y3)

    return pl.pallas_call(
        body,
        grid=(_E // _EBLK,),
        in_specs=[_bspec(_EBLK, _WY), _const_spec((8, _D)), _const_spec((8, _D)),
                  _const_spec((1, _D)), _const_spec((1, _D)),
                  _const_spec((_D, _D)), _const_spec((1, _D)),
                  _const_spec((1, _D)), _const_spec((1, _D)),
                  _const_spec((_D, _D)), _const_spec((1, _D))],
        out_specs=_const_spec((8, _D)),
        out_shape=jax.ShapeDtypeStruct((8, _D), _f32),
    )(Y, s1, s2, g1, be1, w2t, b2, g2, be2, pw1t, pb1)


def _edge_pass_e(Y, s1, s2, s3, g1, be1, w2t, b2, g2, be2,
                 pw1t, pb1, g3, be3, pw2t, pb2):
    def body(y_ref, s1_ref, s2_ref, s3_ref, g1_r, be1_r, w2_r, b2_r,
             g2_r, be2_r, pw1_r, pb1_r, g3_r, be3_r, pw2_r, pb2_r, o_ref):
        sc1, sh1 = _bn_affine(s1_ref[...], g1_r[...], be1_r[...], float(_E))
        z1 = jnp.maximum(y_ref[:, :_D] * sc1 + sh1, 0.0)
        y2 = jnp.dot(z1, w2_r[...], preferred_element_type=_f32) + b2_r[...]
        sc2, sh2 = _bn_affine(s2_ref[...], g2_r[...], be2_r[...], float(_E))
        z2 = jnp.maximum(y2 * sc2 + sh2, 0.0)
        y3 = jnp.dot(z2, pw1_r[...], preferred_element_type=_f32) + pb1_r[...]
        sc3, sh3 = _bn_affine(s3_ref[...], g3_r[...], be3_r[...], float(_E))
        z3 = jnp.maximum(y3 * sc3 + sh3, 0.0)
        pw = jnp.dot(z3, pw2_r[...], preferred_element_type=_f32) + pb2_r[...]
        wpos = y_ref[:, _D:_D + 3] * pw
        o_ref[...] = jnp.concatenate(
            [z2, wpos, jnp.ones((_EBLK, 1), _f32),
             jnp.zeros((_EBLK, _W - _D - 4), _f32)], axis=1)

    return pl.pallas_call(
        body,
        grid=(_E // _EBLK,),
        in_specs=[_bspec(_EBLK, _WY),
                  _const_spec((8, _D)), _const_spec((8, _D)),
                  _const_spec((8, _D)),
                  _const_spec((1, _D)), _const_spec((1, _D)),
                  _const_spec((_D, _D)), _const_spec((1, _D)),
                  _const_spec((1, _D)), _const_spec((1, _D)),
                  _const_spec((_D, _D)), _const_spec((1, _D)),
                  _const_spec((1, _D)), _const_spec((1, _D)),
                  _const_spec((_D, 1)), _const_spec((1, 1))],
        out_specs=_bspec(_EBLK, _W),
        out_shape=jax.ShapeDtypeStruct((_E, _W), _f32),
    )(Y, s1, s2, s3, g1, be1, w2t, b2, g2, be2, pw1t, pb1, g3, be3, pw2t, pb2)


def _node_pass_1(T, A, u1at, u1bt, ub1):
    def body(t_ref, a_ref, wa, wb, b, s_ref):
        h = t_ref[:, :_D]
        denom = jnp.maximum(a_ref[:, _D + 3:_D + 4], 1.0)
        magg = a_ref[:, :_D] / denom
        u1 = (jnp.dot(h, wa[...], preferred_element_type=_f32)
              + jnp.dot(magg, wb[...], preferred_element_type=_f32) + b[...])
        _acc_stats(s_ref, u1)

    return pl.pallas_call(
        body,
        grid=(_N // _NBLK,),
        in_specs=[_bspec(_NBLK, _W), _bspec(_NBLK, _W),
                  _const_spec((_D, _D)), _const_spec((_D, _D)),
                  _const_spec((1, _D))],
        out_specs=_const_spec((8, _D)),
        out_shape=jax.ShapeDtypeStruct((8, _D), _f32),
    )(T, A, u1at, u1bt, ub1)


def _node_pass_2(T, A, s1, u1at, u1bt, ub1, g1, be1, u2t, ub2):
    def body(t_ref, a_ref, s1_ref, wa, wb, b, g1_r, be1_r, w2_r, b2_r, s_ref):
        h = t_ref[:, :_D]
        denom = jnp.maximum(a_ref[:, _D + 3:_D + 4], 1.0)
        magg = a_ref[:, :_D] / denom
        u1 = (jnp.dot(h, wa[...], preferred_element_type=_f32)
              + jnp.dot(magg, wb[...], preferred_element_type=_f32) + b[...])
        sc1, sh1 = _bn_affine(s1_ref[...], g1_r[...], be1_r[...], float(_N))
        z1 = jnp.maximum(u1 * sc1 + sh1, 0.0)
        u2 = jnp.dot(z1, w2_r[...], preferred_element_type=_f32) + b2_r[...]
        _acc_stats(s_ref, u2)

    return pl.pallas_call(
        body,
        grid=(_N // _NBLK,),
        in_specs=[_bspec(_NBLK, _W), _bspec(_NBLK, _W), _const_spec((8, _D)),
                  _const_spec((_D, _D)), _const_spec((_D, _D)),
                  _const_spec((1, _D)), _const_spec((1, _D)),
                  _const_spec((1, _D)), _const_spec((_D, _D)),
                  _const_spec((1, _D))],
        out_specs=_const_spec((8, _D)),
        out_shape=jax.ShapeDtypeStruct((8, _D), _f32),
    )(T, A, s1, u1at, u1bt, ub1, g1, be1, u2t, ub2)


def _node_pass_3(T, A, s1, s2, u1at, u1bt, ub1, g1, be1, u2t, ub2,
                 g2, be2, owt, ob):
    def body(t_ref, a_ref, s1_ref, s2_ref, wa, wb, b, g1_r, be1_r,
             w2_r, b2_r, g2_r, be2_r, ow_r, ob_r, o_ref):
        h = t_ref[:, :_D]
        denom = jnp.maximum(a_ref[:, _D + 3:_D + 4], 1.0)
        magg = a_ref[:, :_D] / denom
        paggr = a_ref[:, _D:_D + 3] / denom
        u1 = (jnp.dot(h, wa[...], preferred_element_type=_f32)
              + jnp.dot(magg, wb[...], preferred_element_type=_f32) + b[...])
        sc1, sh1 = _bn_affine(s1_ref[...], g1_r[...], be1_r[...], float(_N))
        z1 = jnp.maximum(u1 * sc1 + sh1, 0.0)
        u2 = jnp.dot(z1, w2_r[...], preferred_element_type=_f32) + b2_r[...]
        sc2, sh2 = _bn_affine(s2_ref[...], g2_r[...], be2_r[...], float(_N))
        z2 = jnp.maximum(u2 * sc2 + sh2, 0.0)
        h_out = jnp.dot(z2, ow_r[...], preferred_element_type=_f32) + ob_r[...]
        pos_out = t_ref[:, _D:_D + 3] + paggr
        o_ref[...] = jnp.concatenate(
            [h_out, pos_out, jnp.zeros((_NBLK, _W - _D - 3), _f32)], axis=1)

    return pl.pallas_call(
        body,
        grid=(_N // _NBLK,),
        in_specs=[_bspec(_NBLK, _W), _bspec(_NBLK, _W),
                  _const_spec((8, _D)), _const_spec((8, _D)),
                  _const_spec((_D, _D)), _const_spec((_D, _D)),
                  _const_spec((1, _D)), _const_spec((1, _D)),
                  _const_spec((1, _D)), _const_spec((_D, _D)),
                  _const_spec((1, _D)), _const_spec((1, _D)),
                  _const_spec((1, _D)), _const_spec((_D, _D)),
                  _const_spec((1, _D))],
        out_specs=_bspec(_NBLK, _W),
        out_shape=jax.ShapeDtypeStruct((_N, _W), _f32),
    )(T, A, s1, s2, u1at, u1bt, ub1, g1, be1, u2t, ub2, g2, be2, owt, ob)


def _col_sum(T):
    def body(t_ref, s_ref):
        _acc_stats(s_ref, t_ref[:, :_D])

    return pl.pallas_call(
        body,
        grid=(_N // _NBLK,),
        in_specs=[_bspec(_NBLK, _W)],
        out_specs=_const_spec((8, _D)),
        out_shape=jax.ShapeDtypeStruct((8, _D), _f32),
    )(T)


def _heads(hsum_r, hsum_l, wvr, bvr, wor, bor, wvl, bvl, wol, bol,
           frr, frb, ftr, ftb, flr, flb, ftl, ftlb):
    def body(hr_ref, hl_ref, wvr_r, bvr_r, wor_r, bor_r, wvl_r, bvl_r,
             wol_r, bol_r, frr_r, frb_r, ftr_r, ftb_r, flr_r, flb_r,
             ftl_r, ftlb_r, o_ref):
        hr = hr_ref[0:1, :] / float(_N)
        hl = hl_ref[0:1, :] / float(_N)
        ar = (jnp.dot(jnp.dot(hl, wvr_r[...], preferred_element_type=_f32)
                      + bvr_r[...], wor_r[...], preferred_element_type=_f32)
              + bor_r[...])
        al = (jnp.dot(jnp.dot(hr, wvl_r[...], preferred_element_type=_f32)
                      + bvl_r[...], wol_r[...], preferred_element_type=_f32)
              + bol_r[...])
        rr = jnp.dot(ar, frr_r[...], preferred_element_type=_f32) + frb_r[...]
        tr = jnp.dot(ar, ftr_r[...], preferred_element_type=_f32) + ftb_r[...]
        rl = jnp.dot(al, flr_r[...], preferred_element_type=_f32) + flb_r[...]
        tl = jnp.dot(al, ftl_r[...], preferred_element_type=_f32) + ftlb_r[...]
        z4 = jnp.zeros((1, 4), _f32)
        row0 = jnp.concatenate([rr, tr, z4], axis=1)
        row1 = jnp.concatenate([rl, tl, z4], axis=1)
        o_ref[...] = jnp.concatenate(
            [row0, row1, jnp.zeros((6, 16), _f32)], axis=0)

    specs = ([_const_spec((8, _D))] * 2
             + [_const_spec((_D, _D)), _const_spec((1, _D))] * 4
             + [_const_spec((_D, 9)), _const_spec((1, 9)),
                _const_spec((_D, 3)), _const_spec((1, 3))] * 2)
    return pl.pallas_call(
        body,
        grid=(1,),
        in_specs=specs,
        out_specs=_const_spec((8, 16)),
        out_shape=jax.ShapeDtypeStruct((8, 16), _f32),
    )(hsum_r, hsum_l, wvr, bvr, wor, bor, wvl, bvl, wol, bol,
      frr, frb, ftr, ftb, flr, flb, ftl, ftlb)


def _coords(pos, RT, row):
    def body(p_ref, rt_ref, o_ref):
        p = p_ref[...]
        cols = []
        for j in range(3):
            c = (p[:, 0:1] * rt_ref[row, 3 * j]
                 + p[:, 1:2] * rt_ref[row, 3 * j + 1]
                 + p[:, 2:3] * rt_ref[row, 3 * j + 2]
                 + rt_ref[row, 9 + j])
            cols.append(c)
        o_ref[...] = jnp.concatenate(cols, axis=1)

    return pl.pallas_call(
        body,
        grid=(_N // _NBLK,),
        in_specs=[_bspec(_NBLK, 3), _const_spec((8, 16))],
        out_specs=_bspec(_NBLK, 3),
        out_shape=jax.ShapeDtypeStruct((_N, 3), _f32),
    )(pos, RT)


def _prep_layer(p):
    r = lambda a: a.reshape(1, -1)
    return dict(
        w1at=p["msg_W1"][:, :_D].T, w1bt=p["msg_W1"][:, _D:2 * _D].T,
        w1c=r(p["msg_W1"][:, 2 * _D]), b1=r(p["msg_b1"]),
        g1=r(p["msg_g1"]), be1=r(p["msg_be1"]),
        w2t=p["msg_W2"].T, b2=r(p["msg_b2"]),
        g2=r(p["msg_g2"]), be2=r(p["msg_be2"]),
        pw1t=p["pos_W1"].T, pb1=r(p["pos_b1"]),
        g3=r(p["pos_g1"]), be3=r(p["pos_be1"]),
        pw2t=p["pos_W2"].T, pb2=p["pos_b2"].reshape(1, 1),
        u1at=p["upd_W1"][:, :_D].T, u1bt=p["upd_W1"][:, _D:].T,
        ub1=r(p["upd_b1"]), ug1=r(p["upd_g1"]), ube1=r(p["upd_be1"]),
        u2t=p["upd_W2"].T, ub2=r(p["upd_b2"]),
        ug2=r(p["upd_g2"]), ube2=r(p["upd_be2"]),
        owt=p["out_W"].T, ob=r(p["out_b"]),
    )


def _mpnn_layer(T, idx_g, dst, q):
    G = _sc_gather(T, idx_g)
    Y, s1 = _edge_pass_b(G, q["w1at"], q["w1bt"], q["w1c"], q["b1"])
    s2 = _edge_pass_c(Y, s1, q["g1"], q["be1"], q["w2t"], q["b2"])
    s3 = _edge_pass_d(Y, s1, s2, q["g1"], q["be1"], q["w2t"], q["b2"],
                      q["g2"], q["be2"], q["pw1t"], q["pb1"])
    S = _edge_pass_e(Y, s1, s2, s3, q["g1"], q["be1"], q["w2t"], q["b2"],
                     q["g2"], q["be2"], q["pw1t"], q["pb1"],
                     q["g3"], q["be3"], q["pw2t"], q["pb2"])
    A0 = _sc_scatter(S, dst, 0)
    A1 = _sc_scatter(S, dst, 2 * _QTR)
    A = jnp.concatenate(
        [A0[:_QTR], A0[_ACCR:_ACCR + _QTR],
         A1[:_QTR], A1[_ACCR:_ACCR + _QTR]], axis=0)
    n1 = _node_pass_1(T, A, q["u1at"], q["u1bt"], q["ub1"])
    n2 = _node_pass_2(T, A, n1, q["u1at"], q["u1bt"], q["ub1"],
                      q["ug1"], q["ube1"], q["u2t"], q["ub2"])
    return _node_pass_3(T, A, n1, n2, q["u1at"], q["u1bt"], q["ub1"],
                        q["ug1"], q["ube1"], q["u2t"], q["ub2"],
                        q["ug2"], q["ube2"], q["owt"], q["ob"])


def kernel(receptor_x, receptor_pos, ligand_x, ligand_pos, params,
           receptor_edge_index, ligand_edge_index):
    p = params
    r = lambda a: a.reshape(1, -1)

    Tr = _pack_table(receptor_x, receptor_pos,
                     r(p["lin_in_rec_W"][:, 0]), r(p["lin_in_rec_b"]))
    Tl = _pack_table(ligand_x, ligand_pos,
                     r(p["lin_in_lig_W"][:, 0]), r(p["lin_in_lig_b"]))

    rdst = receptor_edge_index[1]
    ridx = jnp.concatenate([rdst, receptor_edge_index[0]])
    ldst = ligand_edge_index[1]
    lidx = jnp.concatenate([ldst, ligand_edge_index[0]])

    q1, q2 = _prep_layer(p["rec_l1"]), _prep_layer(p["rec_l2"])
    q3, q4 = _prep_layer(p["lig_l1"]), _prep_layer(p["lig_l2"])

    Tr = _mpnn_layer(Tr, ridx, rdst, q1)
    Tr = _mpnn_layer(Tr, ridx, rdst, q2)
    Tl = _mpnn_layer(Tl, lidx, ldst, q3)
    Tl = _mpnn_layer(Tl, lidx, ldst, q4)

    hsum_r = _col_sum(Tr)
    hsum_l = _col_sum(Tl)

    ra, la = p["rec_attn"], p["lig_attn"]
    RT = _heads(hsum_r, hsum_l,
                ra["Wv"].T, r(ra["bv"]), ra["Wo"].T, r(ra["bo"]),
                la["Wv"].T, r(la["bv"]), la["Wo"].T, r(la["bo"]),
                p["fc_r_rec_W"].T, r(p["fc_r_rec_b"]),
                p["fc_t_rec_W"].T, r(p["fc_t_rec_b"]),
                p["fc_r_lig_W"].T, r(p["fc_r_lig_b"]),
                p["fc_t_lig_W"].T, r(p["fc_t_lig_b"]))

    rec_coords = _coords(receptor_pos, RT, 0)
    lig_coords = _coords(ligand_pos, RT, 1)
    return (rec_coords, lig_coords)


# speedup vs baseline: 3.7554x; 1.0007x over previous
import functools

import jax
import jax.numpy as jnp
from jax import lax
from jax.experimental import pallas as pl
from jax.experimental.pallas import tpu as pltpu
from jax.experimental.pallas import tpu_sc as plsc

_N = 50000
_E = 800000
_D = 64
_W = 128
_WY = 80
_EBLK = 4000
_NBLK = 2000
_NC = 2
_NS = 16
_QTR = 12500
_ACCR = 12544
_ZROWS = 56
_GOPS = (2 * _E) // 128
_GPW = 392

_f32 = jnp.float32


def _sc_gather(table, idx):
    mesh = plsc.VectorSubcoreMesh(core_axis_name="c", subcore_axis_name="s")

    @functools.partial(
        pl.kernel,
        out_type=jax.ShapeDtypeStruct((2 * _E, _W), _f32),
        mesh=mesh,
        scratch_types=[
            pltpu.VMEM((128,), jnp.int32),
            pltpu.VMEM((128,), jnp.int32),
            pltpu.VMEM((128, _W), _f32),
            pltpu.VMEM((128, _W), _f32),
            pltpu.SemaphoreType.DMA,
            pltpu.SemaphoreType.DMA,
        ],
    )
    def gk(table_hbm, idx_hbm, out_hbm, i0, i1, r0, r1, g0, g1):
        wid = lax.axis_index("s") * _NC + lax.axis_index("c")
        ib = (i0, i1)
        rb = (r0, r1)
        gs = (g0, g1)
        base = wid * _GPW
        limit = jnp.minimum(base + _GPW, _GOPS)

        def start(j, b):
            @pl.when(j < limit)
            def _():
                pltpu.sync_copy(idx_hbm.at[pl.ds(j * 128, 128)], ib[b])
                pltpu.async_copy(table_hbm.at[ib[b]], rb[b], gs[b])

        def finish(j, b):
            @pl.when(j < limit)
            def _():
                pltpu.make_async_copy(table_hbm.at[ib[b]], rb[b], gs[b]).wait()
                pltpu.sync_copy(rb[b], out_hbm.at[pl.ds(j * 128, 128)])
        start(base, 0)
        start(base + 1, 1)

        def body(g, carry):
            j = base + 2 * g
            finish(j, 0)
            start(j + 2, 0)
            finish(j + 1, 1)
            start(j + 3, 1)
            return carry

        lax.fori_loop(0, _GPW // 2, body, 0)

    return gk(table, idx)


def _sc_scatter(payload, dst, nbase):
    mesh = plsc.VectorSubcoreMesh(core_axis_name="c", subcore_axis_name="s")

    @functools.partial(
        pl.kernel,
        out_type=jax.ShapeDtypeStruct((2 * _ACCR, _W), _f32),
        mesh=mesh,
        scratch_types=[
            pltpu.VMEM((80,), jnp.int32),
            pltpu.VMEM((80,), jnp.int32),
            pltpu.VMEM((8, 80), jnp.int32),
            pltpu.VMEM((8, 80), jnp.int32),
            pltpu.VMEM((80, _W), _f32),
            pltpu.VMEM((80, _W), _f32),
            pltpu.VMEM((_ZROWS, _W), _f32),
            pltpu.VMEM_SHARED((_ACCR, _W), _f32),
            pltpu.SemaphoreType.DMA,
            pltpu.SemaphoreType.DMA,
            pltpu.SemaphoreType.DMA,
            pltpu.SemaphoreType.DMA,
        ],
    )
    def sk(pay_hbm, dst_hbm, out_hbm, d0, d1, l0, l1, r0, r1, zbuf, acc,
           sd0, sd1, sp0, sp1):
        c = lax.axis_index("c")
        s = lax.axis_index("s")
        lo = nbase + c * _QTR
        hi = lo + _QTR
        db = (d0, d1)
        lb = (l0, l1)
        rb = (r0, r1)
        sd = (sd0, sd1)
        sp = (sp0, sp1)

        def zrow(rr, carry):
            for k in range(_W // 16):
                zbuf[rr, pl.ds(k * 16, 16)] = jnp.zeros((16,), _f32)
            return carry

        lax.fori_loop(0, _ZROWS, zrow, 0)
        for k in range(_ACCR // _NS // _ZROWS):
            pltpu.sync_copy(
                zbuf,
                acc.at[pl.ds(s * (_ACCR // _NS) + k * _ZROWS, _ZROWS)])
        plsc.subcore_barrier()

        nchunk = (_E // _NS) // 80
        tbase = s * (_E // _NS)
        iota = lax.broadcasted_iota(jnp.int32, (16,), 0)

        def start(i, b):
            @pl.when(i < nchunk)
            def _():
                base = tbase + i * 80
                pltpu.async_copy(dst_hbm.at[pl.ds(base, 80)], db[b], sd[b])
                pltpu.async_copy(pay_hbm.at[pl.ds(base, 80)], rb[b], sp[b])

        def finish(i, b):
            @pl.when(i < nchunk)
            def _():
                base = tbase + i * 80
                pltpu.make_async_copy(
                    dst_hbm.at[pl.ds(base, 80)], db[b], sd[b]).wait()
                for k in range(5):
                    v = db[b][pl.ds(k * 16, 16)]
                    m = (v >= lo) & (v < hi)
                    li = jnp.where(m, v - lo, _QTR + (iota & 7))
                    lb[b][0, pl.ds(k * 16, 16)] = li
                pltpu.make_async_copy(
                    pay_hbm.at[pl.ds(base, 80)], rb[b], sp[b]).wait()
                pltpu.sync_copy(rb[b], acc.at[lb[b].at[0]], add=True)

        start(0, 0)
        start(1, 1)

        def pair(g, carry):
            i = 2 * g
            finish(i, 0)
            start(i + 2, 0)
            finish(i + 1, 1)
            start(i + 3, 1)
            return carry

        lax.fori_loop(0, (nchunk + 1) // 2, pair, 0)
        plsc.subcore_barrier()

        span = _ACCR // _NS
        pltpu.sync_copy(acc.at[pl.ds(s * span, span)],
                        out_hbm.at[pl.ds(c * _ACCR + s * span, span)])

    return sk(payload, dst)


def _bspec(blk, w):
    return pl.BlockSpec((blk, w), lambda i: (i, 0))


def _const_spec(shape):
    return pl.BlockSpec(shape, lambda i: (0, 0))


def _bn_affine(stats, g, be, n):
    s = stats[0:1, :]
    q = stats[1:2, :]
    m = s / n
    v = q / n - m * m
    scale = g * lax.rsqrt(v + 1e-5)
    return scale, be - m * scale


def _acc_stats(ref, y):
    upd = jnp.concatenate(
        [jnp.sum(y, axis=0, keepdims=True),
         jnp.sum(y * y, axis=0, keepdims=True),
         jnp.zeros((6, _D), _f32)], axis=0)

    @pl.when(pl.program_id(0) == 0)
    def _():
        ref[...] = jnp.zeros_like(ref)

    ref[...] += upd


def _pack_table(x, pos, w_in, b_in):
    def body(x_ref, p_ref, w_ref, b_ref, o_ref):
        h = x_ref[...] * w_ref[...] + b_ref[...]
        o_ref[...] = jnp.concatenate(
            [h, p_ref[...], jnp.zeros((_NBLK, _W - _D - 3), _f32)], axis=1)

    return pl.pallas_call(
        body,
        grid=(_N // _NBLK,),
        in_specs=[_bspec(_NBLK, 1), _bspec(_NBLK, 3),
                  _const_spec((1, _D)), _const_spec((1, _D))],
        out_specs=_bspec(_NBLK, _W),
        out_shape=jax.ShapeDtypeStruct((_N, _W), _f32),
    )(x, pos, w_in, b_in)


def _edge_pass_b(G, w1at, w1bt, w1c, b1):
    def body(gd, gs, wa, wb, wc, b, y_ref, s_ref):
        hd = gd[:, :_D]
        hs = gs[:, :_D]
        pd = gd[:, _D:_D + 3] - gs[:, _D:_D + 3]
        dist = jnp.sqrt(jnp.sum(pd * pd, axis=1, keepdims=True))
        y1 = (jnp.dot(hd, wa[...], preferred_element_type=_f32)
              + jnp.dot(hs, wb[...], preferred_element_type=_f32)
              + dist * wc[...] + b[...])
        y_ref[...] = jnp.concatenate(
            [y1, pd, dist, jnp.zeros((_EBLK, _WY - _D - 4), _f32)], axis=1)
        _acc_stats(s_ref, y1)

    nblk = _E // _EBLK
    return pl.pallas_call(
        body,
        grid=(nblk,),
        in_specs=[
            pl.BlockSpec((_EBLK, _W), lambda i: (i, 0)),
            pl.BlockSpec((_EBLK, _W), lambda i: (i + nblk, 0)),
            _const_spec((_D, _D)), _const_spec((_D, _D)),
            _const_spec((1, _D)), _const_spec((1, _D)),
        ],
        out_specs=[_bspec(_EBLK, _WY), _const_spec((8, _D))],
        out_shape=[jax.ShapeDtypeStruct((_E, _WY), _f32),
                   jax.ShapeDtypeStruct((8, _D), _f32)],
    )(G, G, w1at, w1bt, w1c, b1)


def _edge_pass_c(Y, s1, g1, be1, w2t, b2):
    def body(y_ref, s1_ref, g1_r, be1_r, w2_r, b2_r, s_ref):
        sc1, sh1 = _bn_affine(s1_ref[...], g1_r[...], be1_r[...], float(_E))
        z1 = jnp.maximum(y_ref[:, :_D] * sc1 + sh1, 0.0)
        y2 = jnp.dot(z1, w2_r[...], preferred_element_type=_f32) + b2_r[...]
        _acc_stats(s_ref, y2)

    return pl.pallas_call(
        body,
        grid=(_E // _EBLK,),
        in_specs=[_bspec(_EBLK, _WY), _const_spec((8, _D)),
                  _const_spec((1, _D)), _const_spec((1, _D)),
                  _const_spec((_D, _D)), _const_spec((1, _D))],
        out_specs=_const_spec((8, _D)),
        out_shape=jax.ShapeDtypeStruct((8, _D), _f32),
    )(Y, s1, g1, be1, w2t, b2)


def _edge_pass_d(Y, s1, s2, g1, be1, w2t, b2, g2, be2, pw1t, pb1):
    def body(y_ref, s1_ref, s2_ref, g1_r, be1_r, w2_r, b2_r,
             g2_r, be2_r, pw1_r, pb1_r, s_ref):
        sc1, sh1 = _bn_affine(s1_ref[...], g1_r[...], be1_r[...], float(_E))
        z1 = jnp.maximum(y_ref[:, :_D] * sc1 + sh1, 0.0)
        y2 = jnp.dot(z1, w2_r[...], preferred_element_type=_f32) + b2_r[...]
        sc2, sh2 = _bn_affine(s2_ref[...], g2_r[...], be2_r[...], float(_E))
        z2 = jnp.maximum(y2 * sc2 + sh2, 0.0)
        y3 = jnp.dot(z2, pw1_r[...], preferred_element_type=_f32) + pb1_r[...]
        _acc_stats(s_ref, y3)

    return pl.pallas_call(
        body,
        grid=(_E // _EBLK,),
        in_specs=[_bspec(_EBLK, _WY), _const_spec((8, _D)), _const_spec((8, _D)),
                  _const_spec((1, _D)), _const_spec((1, _D)),
                  _const_spec((_D, _D)), _const_spec((1, _D)),
                  _const_spec((1, _D)), _const_spec((1, _D)),
                  _const_spec((_D, _D)), _const_spec((1, _D))],
        out_specs=_const_spec((8, _D)),
        out_shape=jax.ShapeDtypeStruct((8, _D), _f32),
    )(Y, s1, s2, g1, be1, w2t, b2, g2, be2, pw1t, pb1)


def _edge_pass_e(Y, s1, s2, s3, g1, be1, w2t, b2, g2, be2,
                 pw1t, pb1, g3, be3, pw2t, pb2):
    def body(y_ref, s1_ref, s2_ref, s3_ref, g1_r, be1_r, w2_r, b2_r,
             g2_r, be2_r, pw1_r, pb1_r, g3_r, be3_r, pw2_r, pb2_r, o_ref):
        sc1, sh1 = _bn_affine(s1_ref[...], g1_r[...], be1_r[...], float(_E))
        z1 = jnp.maximum(y_ref[:, :_D] * sc1 + sh1, 0.0)
        y2 = jnp.dot(z1, w2_r[...], preferred_element_type=_f32) + b2_r[...]
        sc2, sh2 = _bn_affine(s2_ref[...], g2_r[...], be2_r[...], float(_E))
        z2 = jnp.maximum(y2 * sc2 + sh2, 0.0)
        y3 = jnp.dot(z2, pw1_r[...], preferred_element_type=_f32) + pb1_r[...]
        sc3, sh3 = _bn_affine(s3_ref[...], g3_r[...], be3_r[...], float(_E))
        z3 = jnp.maximum(y3 * sc3 + sh3, 0.0)
        pw = jnp.dot(z3, pw2_r[...], preferred_element_type=_f32) + pb2_r[...]
        wpos = y_ref[:, _D:_D + 3] * pw
        o_ref[...] = jnp.concatenate(
            [z2, wpos, jnp.ones((_EBLK, 1), _f32),
             jnp.zeros((_EBLK, _W - _D - 4), _f32)], axis=1)

    return pl.pallas_call(
        body,
        grid=(_E // _EBLK,),
        in_specs=[_bspec(_EBLK, _WY),
                  _const_spec((8, _D)), _const_spec((8, _D)),
                  _const_spec((8, _D)),
                  _const_spec((1, _D)), _const_spec((1, _D)),
                  _const_spec((_D, _D)), _const_spec((1, _D)),
                  _const_spec((1, _D)), _const_spec((1, _D)),
                  _const_spec((_D, _D)), _const_spec((1, _D)),
                  _const_spec((1, _D)), _const_spec((1, _D)),
                  _const_spec((_D, 1)), _const_spec((1, 1))],
        out_specs=_bspec(_EBLK, _W),
        out_shape=jax.ShapeDtypeStruct((_E, _W), _f32),
    )(Y, s1, s2, s3, g1, be1, w2t, b2, g2, be2, pw1t, pb1, g3, be3, pw2t, pb2)


def _node_pass_1(T, A, u1at, u1bt, ub1):
    def body(t_ref, a_ref, wa, wb, b, s_ref):
        h = t_ref[:, :_D]
        denom = jnp.maximum(a_ref[:, _D + 3:_D + 4], 1.0)
        magg = a_ref[:, :_D] / denom
        u1 = (jnp.dot(h, wa[...], preferred_element_type=_f32)
              + jnp.dot(magg, wb[...], preferred_element_type=_f32) + b[...])
        _acc_stats(s_ref, u1)

    return pl.pallas_call(
        body,
        grid=(_N // _NBLK,),
        in_specs=[_bspec(_NBLK, _W), _bspec(_NBLK, _W),
                  _const_spec((_D, _D)), _const_spec((_D, _D)),
                  _const_spec((1, _D))],
        out_specs=_const_spec((8, _D)),
        out_shape=jax.ShapeDtypeStruct((8, _D), _f32),
    )(T, A, u1at, u1bt, ub1)


def _node_pass_2(T, A, s1, u1at, u1bt, ub1, g1, be1, u2t, ub2):
    def body(t_ref, a_ref, s1_ref, wa, wb, b, g1_r, be1_r, w2_r, b2_r, s_ref):
        h = t_ref[:, :_D]
        denom = jnp.maximum(a_ref[:, _D + 3:_D + 4], 1.0)
        magg = a_ref[:, :_D] / denom
        u1 = (jnp.dot(h, wa[...], preferred_element_type=_f32)
              + jnp.dot(magg, wb[...], preferred_element_type=_f32) + b[...])
        sc1, sh1 = _bn_affine(s1_ref[...], g1_r[...], be1_r[...], float(_N))
        z1 = jnp.maximum(u1 * sc1 + sh1, 0.0)
        u2 = jnp.dot(z1, w2_r[...], preferred_element_type=_f32) + b2_r[...]
        _acc_stats(s_ref, u2)

    return pl.pallas_call(
        body,
        grid=(_N // _NBLK,),
        in_specs=[_bspec(_NBLK, _W), _bspec(_NBLK, _W), _const_spec((8, _D)),
                  _const_spec((_D, _D)), _const_spec((_D, _D)),
                  _const_spec((1, _D)), _const_spec((1, _D)),
                  _const_spec((1, _D)), _const_spec((_D, _D)),
                  _const_spec((1, _D))],
        out_specs=_const_spec((8, _D)),
        out_shape=jax.ShapeDtypeStruct((8, _D), _f32),
    )(T, A, s1, u1at, u1bt, ub1, g1, be1, u2t, ub2)


def _node_pass_3(T, A, s1, s2, u1at, u1bt, ub1, g1, be1, u2t, ub2,
                 g2, be2, owt, ob):
    def body(t_ref, a_ref, s1_ref, s2_ref, wa, wb, b, g1_r, be1_r,
             w2_r, b2_r, g2_r, be2_r, ow_r, ob_r, o_ref):
        h = t_ref[:, :_D]
        denom = jnp.maximum(a_ref[:, _D + 3:_D + 4], 1.0)
        magg = a_ref[:, :_D] / denom
        paggr = a_ref[:, _D:_D + 3] / denom
        u1 = (jnp.dot(h, wa[...], preferred_element_type=_f32)
              + jnp.dot(magg, wb[...], preferred_element_type=_f32) + b[...])
        sc1, sh1 = _bn_affine(s1_ref[...], g1_r[...], be1_r[...], float(_N))
        z1 = jnp.maximum(u1 * sc1 + sh1, 0.0)
        u2 = jnp.dot(z1, w2_r[...], preferred_element_type=_f32) + b2_r[...]
        sc2, sh2 = _bn_affine(s2_ref[...], g2_r[...], be2_r[...], float(_N))
        z2 = jnp.maximum(u2 * sc2 + sh2, 0.0)
        h_out = jnp.dot(z2, ow_r[...], preferred_element_type=_f32) + ob_r[...]
        pos_out = t_ref[:, _D:_D + 3] + paggr
        o_ref[...] = jnp.concatenate(
            [h_out, pos_out, jnp.zeros((_NBLK, _W - _D - 3), _f32)], axis=1)

    return pl.pallas_call(
        body,
        grid=(_N // _NBLK,),
        in_specs=[_bspec(_NBLK, _W), _bspec(_NBLK, _W),
                  _const_spec((8, _D)), _const_spec((8, _D)),
                  _const_spec((_D, _D)), _const_spec((_D, _D)),
                  _const_spec((1, _D)), _const_spec((1, _D)),
                  _const_spec((1, _D)), _const_spec((_D, _D)),
                  _const_spec((1, _D)), _const_spec((1, _D)),
                  _const_spec((1, _D)), _const_spec((_D, _D)),
                  _const_spec((1, _D))],
        out_specs=_bspec(_NBLK, _W),
        out_shape=jax.ShapeDtypeStruct((_N, _W), _f32),
    )(T, A, s1, s2, u1at, u1bt, ub1, g1, be1, u2t, ub2, g2, be2, owt, ob)


def _col_sum(T):
    def body(t_ref, s_ref):
        _acc_stats(s_ref, t_ref[:, :_D])

    return pl.pallas_call(
        body,
        grid=(_N // _NBLK,),
        in_specs=[_bspec(_NBLK, _W)],
        out_specs=_const_spec((8, _D)),
        out_shape=jax.ShapeDtypeStruct((8, _D), _f32),
    )(T)


def _heads(hsum_r, hsum_l, wvr, bvr, wor, bor, wvl, bvl, wol, bol,
           frr, frb, ftr, ftb, flr, flb, ftl, ftlb):
    def body(hr_ref, hl_ref, wvr_r, bvr_r, wor_r, bor_r, wvl_r, bvl_r,
             wol_r, bol_r, frr_r, frb_r, ftr_r, ftb_r, flr_r, flb_r,
             ftl_r, ftlb_r, o_ref):
        hr = hr_ref[0:1, :] / float(_N)
        hl = hl_ref[0:1, :] / float(_N)
        ar = (jnp.dot(jnp.dot(hl, wvr_r[...], preferred_element_type=_f32)
                      + bvr_r[...], wor_r[...], preferred_element_type=_f32)
              + bor_r[...])
        al = (jnp.dot(jnp.dot(hr, wvl_r[...], preferred_element_type=_f32)
                      + bvl_r[...], wol_r[...], preferred_element_type=_f32)
              + bol_r[...])
        rr = jnp.dot(ar, frr_r[...], preferred_element_type=_f32) + frb_r[...]
        tr = jnp.dot(ar, ftr_r[...], preferred_element_type=_f32) + ftb_r[...]
        rl = jnp.dot(al, flr_r[...], preferred_element_type=_f32) + flb_r[...]
        tl = jnp.dot(al, ftl_r[...], preferred_element_type=_f32) + ftlb_r[...]
        z4 = jnp.zeros((1, 4), _f32)
        row0 = jnp.concatenate([rr, tr, z4], axis=1)
        row1 = jnp.concatenate([rl, tl, z4], axis=1)
        o_ref[...] = jnp.concatenate(
            [row0, row1, jnp.zeros((6, 16), _f32)], axis=0)

    specs = ([_const_spec((8, _D))] * 2
             + [_const_spec((_D, _D)), _const_spec((1, _D))] * 4
             + [_const_spec((_D, 9)), _const_spec((1, 9)),
                _const_spec((_D, 3)), _const_spec((1, 3))] * 2)
    return pl.pallas_call(
        body,
        grid=(1,),
        in_specs=specs,
        out_specs=_const_spec((8, 16)),
        out_shape=jax.ShapeDtypeStruct((8, 16), _f32),
    )(hsum_r, hsum_l, wvr, bvr, wor, bor, wvl, bvl, wol, bol,
      frr, frb, ftr, ftb, flr, flb, ftl, ftlb)


def _coords(pos, RT, row):
    def body(p_ref, rt_ref, o_ref):
        p = p_ref[...]
        cols = []
        for j in range(3):
            c = (p[:, 0:1] * rt_ref[row, 3 * j]
                 + p[:, 1:2] * rt_ref[row, 3 * j + 1]
                 + p[:, 2:3] * rt_ref[row, 3 * j + 2]
                 + rt_ref[row, 9 + j])
            cols.append(c)
        o_ref[...] = jnp.concatenate(cols, axis=1)

    return pl.pallas_call(
        body,
        grid=(_N // _NBLK,),
        in_specs=[_bspec(_NBLK, 3), _const_spec((8, 16))],
        out_specs=_bspec(_NBLK, 3),
        out_shape=jax.ShapeDtypeStruct((_N, 3), _f32),
    )(pos, RT)


def _prep_layer(p):
    r = lambda a: a.reshape(1, -1)
    return dict(
        w1at=p["msg_W1"][:, :_D].T, w1bt=p["msg_W1"][:, _D:2 * _D].T,
        w1c=r(p["msg_W1"][:, 2 * _D]), b1=r(p["msg_b1"]),
        g1=r(p["msg_g1"]), be1=r(p["msg_be1"]),
        w2t=p["msg_W2"].T, b2=r(p["msg_b2"]),
        g2=r(p["msg_g2"]), be2=r(p["msg_be2"]),
        pw1t=p["pos_W1"].T, pb1=r(p["pos_b1"]),
        g3=r(p["pos_g1"]), be3=r(p["pos_be1"]),
        pw2t=p["pos_W2"].T, pb2=p["pos_b2"].reshape(1, 1),
        u1at=p["upd_W1"][:, :_D].T, u1bt=p["upd_W1"][:, _D:].T,
        ub1=r(p["upd_b1"]), ug1=r(p["upd_g1"]), ube1=r(p["upd_be1"]),
        u2t=p["upd_W2"].T, ub2=r(p["upd_b2"]),
        ug2=r(p["upd_g2"]), ube2=r(p["upd_be2"]),
        owt=p["out_W"].T, ob=r(p["out_b"]),
    )


def _mpnn_layer(T, idx_g, dst, q):
    G = _sc_gather(T, idx_g)
    Y, s1 = _edge_pass_b(G, q["w1at"], q["w1bt"], q["w1c"], q["b1"])
    s2 = _edge_pass_c(Y, s1, q["g1"], q["be1"], q["w2t"], q["b2"])
    s3 = _edge_pass_d(Y, s1, s2, q["g1"], q["be1"], q["w2t"], q["b2"],
                      q["g2"], q["be2"], q["pw1t"], q["pb1"])
    S = _edge_pass_e(Y, s1, s2, s3, q["g1"], q["be1"], q["w2t"], q["b2"],
                     q["g2"], q["be2"], q["pw1t"], q["pb1"],
                     q["g3"], q["be3"], q["pw2t"], q["pb2"])
    A0 = _sc_scatter(S, dst, 0)
    A1 = _sc_scatter(S, dst, 2 * _QTR)
    A = jnp.concatenate(
        [A0[:_QTR], A0[_ACCR:_ACCR + _QTR],
         A1[:_QTR], A1[_ACCR:_ACCR + _QTR]], axis=0)
    n1 = _node_pass_1(T, A, q["u1at"], q["u1bt"], q["ub1"])
    n2 = _node_pass_2(T, A, n1, q["u1at"], q["u1bt"], q["ub1"],
                      q["ug1"], q["ube1"], q["u2t"], q["ub2"])
    return _node_pass_3(T, A, n1, n2, q["u1at"], q["u1bt"], q["ub1"],
                        q["ug1"], q["ube1"], q["u2t"], q["ub2"],
                        q["ug2"], q["ube2"], q["owt"], q["ob"])


def kernel(receptor_x, receptor_pos, ligand_x, ligand_pos, params,
           receptor_edge_index, ligand_edge_index):
    p = params
    r = lambda a: a.reshape(1, -1)

    Tr = _pack_table(receptor_x, receptor_pos,
                     r(p["lin_in_rec_W"][:, 0]), r(p["lin_in_rec_b"]))
    Tl = _pack_table(ligand_x, ligand_pos,
                     r(p["lin_in_lig_W"][:, 0]), r(p["lin_in_lig_b"]))

    rdst = receptor_edge_index[1]
    ridx = jnp.concatenate([rdst, receptor_edge_index[0]])
    ldst = ligand_edge_index[1]
    lidx = jnp.concatenate([ldst, ligand_edge_index[0]])

    q1, q2 = _prep_layer(p["rec_l1"]), _prep_layer(p["rec_l2"])
    q3, q4 = _prep_layer(p["lig_l1"]), _prep_layer(p["lig_l2"])

    Tr = _mpnn_layer(Tr, ridx, rdst, q1)
    Tl = _mpnn_layer(Tl, lidx, ldst, q3)
    Tr = _mpnn_layer(Tr, ridx, rdst, q2)
    Tl = _mpnn_layer(Tl, lidx, ldst, q4)

    hsum_r = _col_sum(Tr)
    hsum_l = _col_sum(Tl)

    ra, la = p["rec_attn"], p["lig_attn"]
    RT = _heads(hsum_r, hsum_l,
                ra["Wv"].T, r(ra["bv"]), ra["Wo"].T, r(ra["bo"]),
                la["Wv"].T, r(la["bv"]), la["Wo"].T, r(la["bo"]),
                p["fc_r_rec_W"].T, r(p["fc_r_rec_b"]),
                p["fc_t_rec_W"].T, r(p["fc_t_rec_b"]),
                p["fc_r_lig_W"].T, r(p["fc_r_lig_b"]),
                p["fc_t_lig_W"].T, r(p["fc_t_lig_b"]))

    rec_coords = _coords(receptor_pos, RT, 0)
    lig_coords = _coords(ligand_pos, RT, 1)
    return (rec_coords, lig_coords)
